# Initial kernel scaffold; baseline (speedup 1.0000x reference)
import jax
import jax.numpy as jnp
from jax.experimental import pallas as pl


def kernel(e_ids, e_mask, x_graph, edge_index, batch_idx, data_mask, ft_table, W_text, b_text, W1, b1, W2, b2):
    raise NotImplementedError("write your pallas kernel here")


# speedup vs baseline: 8.3631x; 8.3631x over previous
import functools

import jax
import jax.numpy as jnp
from jax import lax
from jax.experimental import pallas as pl
from jax.experimental.pallas import tpu as pltpu
from jax.experimental.pallas import tpu_sc as plsc

NN = 10000
NE = 320000
D = 128
BB = 1024
LL = 128
NC, NS = 2, 16
NW = NC * NS
CH = 80
EP = NW * CH * 128
NACC = 10240
SLICE = NACC // NS
NPAD = NN + 8
DEGW = 10016

_mesh = functools.partial(plsc.VectorSubcoreMesh,
                          core_axis_name="c", subcore_axis_name="s")

_f32 = jnp.float32
_i32 = jnp.int32


@functools.partial(
    pl.kernel,
    out_type=jax.ShapeDtypeStruct((NW * DEGW,), _f32),
    mesh=_mesh(),
    compiler_params=pltpu.CompilerParams(needs_layout_passes=False),
    scratch_types=[
        pltpu.VMEM((CH, 128), _i32),
        pltpu.VMEM((DEGW,), _f32),
    ],
)
def _deg_kernel(dstp_hbm, out_hbm, dstbuf, acc):
    w = lax.axis_index("c") * NS + lax.axis_index("s")
    pltpu.sync_copy(dstp_hbm.at[pl.ds(w * CH, CH)], dstbuf)
    zero = jnp.zeros((16,), _f32)

    def zbody(i, _):
        acc[pl.ds(i * 16, 16)] = zero
        return 0

    lax.fori_loop(0, DEGW // 16, zbody, 0)
    ones = jnp.ones((16,), _f32)

    def body(i, _):
        idx = dstbuf[i // 8, pl.ds((i % 8) * 16, 16)]
        plsc.addupdate_scatter(acc, [idx], ones)
        return 0

    lax.fori_loop(0, (CH * 128) // 16, body, 0)
    pltpu.sync_copy(acc, out_hbm.at[pl.ds(w * DEGW, DEGW)])


def _scatter_body(table_hbm, srcp_hbm, dstp_hbm, eids_hbm, ftab_hbm,
                  p_hbm, tsum_hbm, idx_s, idx_d, rows, tidv, tacc, sem, accS):
    c = lax.axis_index("c")
    s = lax.axis_index("s")
    w = c * NS + s
    zero = jnp.zeros((16,), _f32)

    def zbody(i, _):
        rows[i // 8, pl.ds((i % 8) * 16, 16)] = zero
        return 0

    lax.fori_loop(0, 128 * 8, zbody, 0)
    for m in range(SLICE // 128):
        pltpu.sync_copy(rows, accS.at[pl.ds(s * SLICE + m * 128, 128)])

    plsc.subcore_barrier()

    pltpu.sync_copy(srcp_hbm.at[pl.ds(w * CH, CH)], idx_s)
    pltpu.sync_copy(dstp_hbm.at[pl.ds(w * CH, CH)], idx_d)

    def ebody(j, _):
        pltpu.async_copy(table_hbm.at[idx_s.at[j]], rows, sem).wait()
        pltpu.sync_copy(rows, accS.at[idx_d.at[j]], add=True)
        return 0

    lax.fori_loop(0, CH, ebody, 0)

    if eids_hbm is not None:
        nseq = BB // NW
        pltpu.sync_copy(eids_hbm.at[pl.ds(w * nseq, nseq)], tidv)

        def tbody(j, _):
            pltpu.async_copy(ftab_hbm.at[tidv.at[j]], rows, sem).wait()

            def rbody(i, carry):
                return tuple(carry[k] + rows[i, pl.ds(k * 16, 16)]
                             for k in range(8))

            accs = lax.fori_loop(
                0, LL, rbody, tuple(jnp.zeros((16,), _f32) for _ in range(8)))
            for k in range(8):
                tacc[j, pl.ds(k * 16, 16)] = accs[k]
            return 0

        lax.fori_loop(0, nseq, tbody, 0)
        pltpu.sync_copy(tacc, tsum_hbm.at[pl.ds(w * nseq, nseq)])

    plsc.subcore_barrier()
    pltpu.sync_copy(accS.at[pl.ds(s * SLICE, SLICE)],
                    p_hbm.at[pl.ds(c * NACC + s * SLICE, SLICE)])


@functools.partial(
    pl.kernel,
    out_type=(jax.ShapeDtypeStruct((NC * NACC, D), _f32),
              jax.ShapeDtypeStruct((BB, D), _f32)),
    mesh=_mesh(),
    compiler_params=pltpu.CompilerParams(needs_layout_passes=False),
    scratch_types=[
        pltpu.VMEM((CH, 128), _i32),
        pltpu.VMEM((CH, 128), _i32),
        pltpu.VMEM((128, D), _f32),
        pltpu.VMEM((BB // NW, LL), _i32),
        pltpu.VMEM((BB // NW, D), _f32),
        pltpu.SemaphoreType.DMA,
        pltpu.VMEM_SHARED((NACC, D), _f32),
    ],
)
def _scatter_text_kernel(table_hbm, srcp_hbm, dstp_hbm, eids_hbm, ftab_hbm,
                         p_hbm, tsum_hbm, idx_s, idx_d, rows, tidv, tacc,
                         sem, accS):
    _scatter_body(table_hbm, srcp_hbm, dstp_hbm, eids_hbm, ftab_hbm,
                  p_hbm, tsum_hbm, idx_s, idx_d, rows, tidv, tacc, sem, accS)


@functools.partial(
    pl.kernel,
    out_type=jax.ShapeDtypeStruct((NC * NACC, D), _f32),
    mesh=_mesh(),
    compiler_params=pltpu.CompilerParams(needs_layout_passes=False),
    scratch_types=[
        pltpu.VMEM((CH, 128), _i32),
        pltpu.VMEM((CH, 128), _i32),
        pltpu.VMEM((128, D), _f32),
        pltpu.SemaphoreType.DMA,
        pltpu.VMEM_SHARED((NACC, D), _f32),
    ],
)
def _scatter_kernel(table_hbm, srcp_hbm, dstp_hbm, p_hbm,
                    idx_s, idx_d, rows, sem, accS):
    _scatter_body(table_hbm, srcp_hbm, dstp_hbm, None, None,
                  p_hbm, None, idx_s, idx_d, rows, None, None, sem, accS)


@functools.partial(
    pl.kernel,
    out_type=jax.ShapeDtypeStruct((BB, D), _f32),
    mesh=_mesh(),
    compiler_params=pltpu.CompilerParams(needs_layout_passes=False),
    scratch_types=[
        pltpu.VMEM((BB // NW,), _i32),
        pltpu.VMEM((BB // NW, D), _f32),
        pltpu.SemaphoreType.DMA,
    ],
)
def _gather_kernel(src_hbm, midx_hbm, out_hbm, idxv, rows, sem):
    w = lax.axis_index("c") * NS + lax.axis_index("s")
    n = BB // NW
    pltpu.sync_copy(midx_hbm.at[pl.ds(w * n, n)], idxv)
    pltpu.async_copy(src_hbm.at[idxv], rows, sem).wait()
    pltpu.sync_copy(rows, out_hbm.at[pl.ds(w * n, n)])


def _prep_body(deg_ref, x_ref, xs0_ref, dinv_ref):
    d = jnp.sum(deg_ref[...], axis=1) + 1.0
    dv = lax.rsqrt(d)
    xs0_ref[...] = x_ref[...] * dv[:, None]
    dinv_ref[...] = jnp.broadcast_to(dv[:, None], dinv_ref.shape)


def _prep_call(degp, x):
    blk = 1000
    return pl.pallas_call(
        _prep_body,
        grid=(NN // blk,),
        in_specs=[pl.BlockSpec((blk, NW), lambda j: (j, 0)),
                  pl.BlockSpec((blk, D), lambda j: (j, 0))],
        out_specs=[pl.BlockSpec((blk, D), lambda j: (j, 0)),
                   pl.BlockSpec((blk, D), lambda j: (j, 0))],
        out_shape=[jax.ShapeDtypeStruct((NN, D), _f32),
                   jax.ShapeDtypeStruct((NN, D), _f32)],
    )(degp, x)


def _mid_body(xs0_ref, p0_ref, p1_ref, dinv_ref, w1_ref, b1_ref, w2_ref,
              out_ref):
    dv = dinv_ref[...]
    agg = (xs0_ref[...] + p0_ref[...] + p1_ref[...]) * dv
    h = jnp.dot(agg, w1_ref[...], preferred_element_type=_f32) + b1_ref[...]
    h = jnp.where(h >= 0, h, 0.01 * h)
    z = jnp.dot(h, w2_ref[...], preferred_element_type=_f32)
    out_ref[...] = z * dv


def _mid_call(xs0, p0, p1, dinvb, W1, b1, W2):
    blk = 1000
    row = pl.BlockSpec((blk, D), lambda j: (j, 0))
    return pl.pallas_call(
        _mid_body,
        grid=(NN // blk,),
        in_specs=[row, row, row, row,
                  pl.BlockSpec((D, 2 * D), lambda j: (0, 0)),
                  pl.BlockSpec((1, 2 * D), lambda j: (0, 0)),
                  pl.BlockSpec((2 * D, D), lambda j: (0, 0))],
        out_specs=row,
        out_shape=jax.ShapeDtypeStruct((NN, D), _f32),
    )(xs0, p0, p1, dinvb, W1, b1, W2)


def _fin_body(xs2_ref, q0_ref, q1_ref, dinv_ref, b2_ref, out_ref):
    blk = out_ref.shape[0]
    j = pl.program_id(0)
    rows = j * blk + lax.broadcasted_iota(_i32, (blk, D), 0)
    v = (xs2_ref[...] + q0_ref[...] + q1_ref[...]) * dinv_ref[...] + b2_ref[...]
    out_ref[...] = jnp.where(rows < NN, v, 0.0)


def _fin_call(xs2, q0, q1, dinvb, b2):
    blk = 1112
    row = pl.BlockSpec((blk, D), lambda j: (j, 0))
    return pl.pallas_call(
        _fin_body,
        grid=(NPAD // blk,),
        in_specs=[row, row, row, row,
                  pl.BlockSpec((1, D), lambda j: (0, 0))],
        out_specs=row,
        out_shape=jax.ShapeDtypeStruct((NPAD, D), _f32),
    )(xs2, q0, q1, dinvb, b2)


def _textmm_body(ts_ref, wt_ref, bt_ref, out_ref):
    t = ts_ref[...] * (1.0 / LL)
    out_ref[...] = (jnp.dot(t, wt_ref[...], preferred_element_type=_f32)
                    + bt_ref[...])


def _textmm_call(tsum, W_text, b_text):
    return pl.pallas_call(
        _textmm_body,
        grid=(1,),
        in_specs=[pl.BlockSpec((BB, D), lambda j: (0, 0)),
                  pl.BlockSpec((D, D), lambda j: (0, 0)),
                  pl.BlockSpec((1, D), lambda j: (0, 0))],
        out_specs=pl.BlockSpec((BB, D), lambda j: (0, 0)),
        out_shape=jax.ShapeDtypeStruct((BB, D), _f32),
    )(tsum, W_text, b_text)


def kernel(e_ids, e_mask, x_graph, edge_index, batch_idx, data_mask,
           ft_table, W_text, b_text, W1, b1, W2, b2):
    x = x_graph.astype(_f32)
    src = edge_index[0].astype(_i32)
    dst = edge_index[1].astype(_i32)
    pad = EP - NE
    srcp = jnp.concatenate([src, jnp.zeros((pad,), _i32)]).reshape(NW * CH, 128)
    dstp = jnp.concatenate([dst, jnp.full((pad,), NN, _i32)]).reshape(NW * CH, 128)

    degp = _deg_kernel(dstp)
    xs0, dinvb = _prep_call(degp.reshape(NW, DEGW).T, x)
    P, tsum = _scatter_text_kernel(xs0, srcp, dstp,
                                   e_ids.astype(_i32), ft_table.astype(_f32))
    xs2 = _mid_call(xs0, P[:NN], P[NACC:NACC + NN], dinvb,
                    W1.astype(_f32), b1.reshape(1, -1).astype(_f32),
                    W2.astype(_f32))
    Q = _scatter_kernel(xs2, srcp, dstp)
    out2p = _fin_call(xs2, Q[:NN], Q[NACC:NACC + NN], dinvb,
                      b2.reshape(1, -1).astype(_f32))
    x_text = _textmm_call(tsum, W_text.astype(_f32),
                          b_text.reshape(1, -1).astype(_f32))
    midx = jnp.where(data_mask, batch_idx.astype(_i32), NN)
    out_graph = _gather_kernel(out2p, midx)
    return (x_text, out_graph)


# speedup vs baseline: 8.5130x; 1.0179x over previous
import functools

import jax
import jax.numpy as jnp
from jax import lax
from jax.experimental import pallas as pl
from jax.experimental.pallas import tpu as pltpu
from jax.experimental.pallas import tpu_sc as plsc

NN = 10000
NE = 320000
D = 128
BB = 1024
LL = 128
NC, NS = 2, 16
NW = NC * NS
CH = 80
EP = NW * CH * 128
NACC = 10240
SLICE = NACC // NS
NPAD = NN + 8
DEGW = 10016

_mesh = functools.partial(plsc.VectorSubcoreMesh,
                          core_axis_name="c", subcore_axis_name="s")

_f32 = jnp.float32
_i32 = jnp.int32


@functools.partial(
    pl.kernel,
    out_type=jax.ShapeDtypeStruct((NW * DEGW,), _f32),
    mesh=_mesh(),
    compiler_params=pltpu.CompilerParams(needs_layout_passes=False),
    scratch_types=[
        pltpu.VMEM((CH * 128,), _i32),
        pltpu.VMEM((DEGW,), _f32),
    ],
)
def _deg_kernel(dstf_hbm, out_hbm, dstbuf, acc):
    w = lax.axis_index("c") * NS + lax.axis_index("s")
    pltpu.sync_copy(dstf_hbm.at[pl.ds(w * CH * 128, CH * 128)], dstbuf)
    zero = jnp.zeros((16,), _f32)

    def zbody(i, _):
        acc[pl.ds(i * 16, 16)] = zero
        return 0

    lax.fori_loop(0, DEGW // 16, zbody, 0)
    ones = jnp.ones((16,), _f32)

    def body(i, _):
        idx = dstbuf[pl.ds(i * 16, 16)]
        plsc.addupdate_scatter(acc, [idx], ones)
        return 0

    lax.fori_loop(0, (CH * 128) // 16, body, 0)
    pltpu.sync_copy(acc, out_hbm.at[pl.ds(w * DEGW, DEGW)])


def _scatter_body(table_hbm, srcp_hbm, dstp_hbm, p_hbm,
                  idx_s, idx_d, rows2, sem, accS):
    c = lax.axis_index("c")
    s = lax.axis_index("s")
    w = c * NS + s
    zero = jnp.zeros((16,), _f32)

    def zbody(i, _):
        rows2[0, i // 8, pl.ds((i % 8) * 16, 16)] = zero
        return 0

    lax.fori_loop(0, 64 * 8, zbody, 0)
    for m in range(SLICE // 64):
        pltpu.sync_copy(rows2.at[0], accS.at[pl.ds(s * SLICE + m * 64, 64)])

    plsc.subcore_barrier()

    pltpu.sync_copy(srcp_hbm.at[pl.ds(w * CH, CH)], idx_s)
    pltpu.sync_copy(dstp_hbm.at[pl.ds(w * CH, CH)], idx_d)

    def gather(j, h, b):
        pltpu.async_copy(table_hbm.at[idx_s.at[j, pl.ds(h * 64, 64)]],
                         rows2.at[b], sem)

    def drain(b):
        pltpu.make_async_copy(table_hbm.at[pl.ds(0, 64)], rows2.at[b],
                              sem).wait()

    def scatter(j, h, b):
        pltpu.sync_copy(rows2.at[b], accS.at[idx_d.at[j, h]], add=True)

    gather(0, 0, 0)

    def pbody(t, _):
        drain(0)
        gather(t, 1, 1)
        scatter(t, 0, 0)
        drain(1)

        @pl.when(t < CH - 1)
        def _g_next():
            gather(t + 1, 0, 0)

        scatter(t, 1, 1)
        return 0

    lax.fori_loop(0, CH, pbody, 0)

    plsc.subcore_barrier()
    pltpu.sync_copy(accS.at[pl.ds(s * SLICE, SLICE)],
                    p_hbm.at[pl.ds(c * NACC + s * SLICE, SLICE)])


@functools.partial(
    pl.kernel,
    out_type=jax.ShapeDtypeStruct((NC * NACC, D), _f32),
    mesh=_mesh(),
    compiler_params=pltpu.CompilerParams(needs_layout_passes=False),
    scratch_types=[
        pltpu.VMEM((CH, 128), _i32),
        pltpu.VMEM((CH, 2, 64), _i32),
        pltpu.VMEM((2, 64, D), _f32),
        pltpu.SemaphoreType.DMA,
        pltpu.VMEM_SHARED((NACC, D), _f32),
    ],
)
def _scatter_kernel(table_hbm, srcp_hbm, dstp_hbm, p_hbm,
                    idx_s, idx_d, rows2, sem, accS):
    _scatter_body(table_hbm, srcp_hbm, dstp_hbm, p_hbm,
                  idx_s, idx_d, rows2, sem, accS)


@functools.partial(
    pl.kernel,
    out_type=jax.ShapeDtypeStruct((BB, D), _f32),
    mesh=_mesh(),
    compiler_params=pltpu.CompilerParams(needs_layout_passes=False),
    scratch_types=[
        pltpu.VMEM((BB // NW, LL), _i32),
        pltpu.VMEM((BB // NW, D), _f32),
        pltpu.VMEM((2, 64, D), _f32),
        pltpu.SemaphoreType.DMA,
    ],
)
def _text_kernel(eids_hbm, ftab_hbm, tsum_hbm, tidv, tacc, rows2, sem):
    w = lax.axis_index("c") * NS + lax.axis_index("s")
    nseq = BB // NW
    pltpu.sync_copy(eids_hbm.at[pl.ds(w * nseq, nseq)], tidv)

    def tgather(j, h, b):
        pltpu.async_copy(ftab_hbm.at[tidv.at[j, pl.ds(h * 64, 64)]],
                         rows2.at[b], sem)

    def twait(b):
        pltpu.make_async_copy(ftab_hbm.at[pl.ds(0, 64)], rows2.at[b],
                              sem).wait()

    def taccum(b, carry):
        def rbody(i, cin):
            return tuple(cin[k] + rows2[b, i, pl.ds(k * 16, 16)]
                         for k in range(8))

        return lax.fori_loop(0, 64, rbody, carry)

    tgather(0, 0, 0)

    def tbody(j, _):
        twait(0)
        tgather(j, 1, 1)
        accs = taccum(0, tuple(jnp.zeros((16,), _f32) for _ in range(8)))
        twait(1)

        @pl.when(j < nseq - 1)
        def _t_next():
            tgather(j + 1, 0, 0)

        accs = taccum(1, accs)
        for k in range(8):
            tacc[j, pl.ds(k * 16, 16)] = accs[k]
        return 0

    lax.fori_loop(0, nseq, tbody, 0)
    pltpu.sync_copy(tacc, tsum_hbm.at[pl.ds(w * nseq, nseq)])


@functools.partial(
    pl.kernel,
    out_type=jax.ShapeDtypeStruct((BB, D), _f32),
    mesh=_mesh(),
    compiler_params=pltpu.CompilerParams(needs_layout_passes=False),
    scratch_types=[
        pltpu.VMEM((BB // NW,), _i32),
        pltpu.VMEM((BB // NW, D), _f32),
        pltpu.SemaphoreType.DMA,
    ],
)
def _gather_kernel(src_hbm, midx_hbm, out_hbm, idxv, rows, sem):
    w = lax.axis_index("c") * NS + lax.axis_index("s")
    n = BB // NW
    pltpu.sync_copy(midx_hbm.at[pl.ds(w * n, n)], idxv)
    pltpu.async_copy(src_hbm.at[idxv], rows, sem).wait()
    pltpu.sync_copy(rows, out_hbm.at[pl.ds(w * n, n)])


def _prep_body(deg_ref, x_ref, xs0_ref, dinv_ref):
    d = jnp.sum(deg_ref[...], axis=1) + 1.0
    dv = lax.rsqrt(d)
    xs0_ref[...] = x_ref[...] * dv[:, None]
    dinv_ref[...] = jnp.broadcast_to(dv[:, None], dinv_ref.shape)


def _prep_call(degp, x):
    blk = 1000
    return pl.pallas_call(
        _prep_body,
        grid=(NN // blk,),
        in_specs=[pl.BlockSpec((blk, NW), lambda j: (j, 0)),
                  pl.BlockSpec((blk, D), lambda j: (j, 0))],
        out_specs=[pl.BlockSpec((blk, D), lambda j: (j, 0)),
                   pl.BlockSpec((blk, D), lambda j: (j, 0))],
        out_shape=[jax.ShapeDtypeStruct((NN, D), _f32),
                   jax.ShapeDtypeStruct((NN, D), _f32)],
    )(degp, x)


def _mid_body(xs0_ref, p0_ref, p1_ref, dinv_ref, w1_ref, b1_ref, w2_ref,
              out_ref):
    dv = dinv_ref[...]
    agg = (xs0_ref[...] + p0_ref[...] + p1_ref[...]) * dv
    h = jnp.dot(agg, w1_ref[...], preferred_element_type=_f32) + b1_ref[...]
    h = jnp.where(h >= 0, h, 0.01 * h)
    z = jnp.dot(h, w2_ref[...], preferred_element_type=_f32)
    out_ref[...] = z * dv


def _mid_call(xs0, p0, p1, dinvb, W1, b1, W2):
    blk = 1000
    row = pl.BlockSpec((blk, D), lambda j: (j, 0))
    return pl.pallas_call(
        _mid_body,
        grid=(NN // blk,),
        in_specs=[row, row, row, row,
                  pl.BlockSpec((D, 2 * D), lambda j: (0, 0)),
                  pl.BlockSpec((1, 2 * D), lambda j: (0, 0)),
                  pl.BlockSpec((2 * D, D), lambda j: (0, 0))],
        out_specs=row,
        out_shape=jax.ShapeDtypeStruct((NN, D), _f32),
    )(xs0, p0, p1, dinvb, W1, b1, W2)


def _fin_body(xs2_ref, q0_ref, q1_ref, dinv_ref, b2_ref, out_ref):
    blk = out_ref.shape[0]
    j = pl.program_id(0)
    rows = j * blk + lax.broadcasted_iota(_i32, (blk, D), 0)
    v = (xs2_ref[...] + q0_ref[...] + q1_ref[...]) * dinv_ref[...] + b2_ref[...]
    out_ref[...] = jnp.where(rows < NN, v, 0.0)


def _fin_call(xs2, q0, q1, dinvb, b2):
    blk = 1112
    row = pl.BlockSpec((blk, D), lambda j: (j, 0))
    return pl.pallas_call(
        _fin_body,
        grid=(NPAD // blk,),
        in_specs=[row, row, row, row,
                  pl.BlockSpec((1, D), lambda j: (0, 0))],
        out_specs=row,
        out_shape=jax.ShapeDtypeStruct((NPAD, D), _f32),
    )(xs2, q0, q1, dinvb, b2)


def _textmm_body(ts_ref, wt_ref, bt_ref, out_ref):
    t = ts_ref[...] * (1.0 / LL)
    out_ref[...] = (jnp.dot(t, wt_ref[...], preferred_element_type=_f32)
                    + bt_ref[...])


def _textmm_call(tsum, W_text, b_text):
    return pl.pallas_call(
        _textmm_body,
        grid=(1,),
        in_specs=[pl.BlockSpec((BB, D), lambda j: (0, 0)),
                  pl.BlockSpec((D, D), lambda j: (0, 0)),
                  pl.BlockSpec((1, D), lambda j: (0, 0))],
        out_specs=pl.BlockSpec((BB, D), lambda j: (0, 0)),
        out_shape=jax.ShapeDtypeStruct((BB, D), _f32),
    )(tsum, W_text, b_text)


def kernel(e_ids, e_mask, x_graph, edge_index, batch_idx, data_mask,
           ft_table, W_text, b_text, W1, b1, W2, b2):
    x = x_graph.astype(_f32)
    src = edge_index[0].astype(_i32)
    dst = edge_index[1].astype(_i32)
    pad = EP - NE
    srcp = jnp.concatenate([src, jnp.zeros((pad,), _i32)]).reshape(NW * CH, 128)
    dstf = jnp.concatenate([dst, jnp.full((pad,), NN, _i32)])
    dstp = dstf.reshape(NW * CH, 2, 64)

    degp = _deg_kernel(dstf)
    xs0, dinvb = _prep_call(degp.reshape(NW, DEGW).T, x)
    P = _scatter_kernel(xs0, srcp, dstp)
    tsum = _text_kernel(e_ids.astype(_i32), ft_table.astype(_f32))
    xs2 = _mid_call(xs0, P[:NN], P[NACC:NACC + NN], dinvb,
                    W1.astype(_f32), b1.reshape(1, -1).astype(_f32),
                    W2.astype(_f32))
    Q = _scatter_kernel(xs2, srcp, dstp)
    out2p = _fin_call(xs2, Q[:NN], Q[NACC:NACC + NN], dinvb,
                      b2.reshape(1, -1).astype(_f32))
    x_text = _textmm_call(tsum, W_text.astype(_f32),
                          b_text.reshape(1, -1).astype(_f32))
    midx = jnp.where(data_mask, batch_idx.astype(_i32), NN)
    out_graph = _gather_kernel(out2p, midx)
    return (x_text, out_graph)


# speedup vs baseline: 9.2465x; 1.0862x over previous
import functools

import jax
import jax.numpy as jnp
from jax import lax
from jax.experimental import pallas as pl
from jax.experimental.pallas import tpu as pltpu
from jax.experimental.pallas import tpu_sc as plsc

NN = 10000
NE = 320000
D = 128
BB = 1024
LL = 128
NC, NS = 2, 16
NW = NC * NS
CH = 80
EP = NW * CH * 128
NACC = 10240
SLICE = NACC // NS
NPAD = NN + 8
DEGW = NACC

_mesh = functools.partial(plsc.VectorSubcoreMesh,
                          core_axis_name="c", subcore_axis_name="s")

_f32 = jnp.float32
_i32 = jnp.int32


@functools.partial(
    pl.kernel,
    out_type=jax.ShapeDtypeStruct((NW * DEGW,), _f32),
    mesh=_mesh(),
    compiler_params=pltpu.CompilerParams(needs_layout_passes=False),
    scratch_types=[
        pltpu.VMEM((CH * 128,), _i32),
        pltpu.VMEM((DEGW,), _f32),
    ],
)
def _deg_kernel(dstf_hbm, out_hbm, dstbuf, acc):
    w = lax.axis_index("c") * NS + lax.axis_index("s")
    pltpu.sync_copy(dstf_hbm.at[pl.ds(w * CH * 128, CH * 128)], dstbuf)
    zero = jnp.zeros((16,), _f32)

    def zbody(i, _):
        acc[pl.ds(i * 16, 16)] = zero
        return 0

    lax.fori_loop(0, DEGW // 16, zbody, 0)
    ones = jnp.ones((16,), _f32)

    def body(i, _):
        idx = dstbuf[pl.ds(i * 16, 16)]
        plsc.addupdate_scatter(acc, [idx], ones)
        return 0

    lax.fori_loop(0, (CH * 128) // 16, body, 0)
    pltpu.sync_copy(acc, out_hbm.at[pl.ds(w * DEGW, DEGW)])


def _scatter_body(table_hbm, srcp_hbm, dstp_hbm, p_hbm,
                  idx_s, idx_d, rows2, sem, accS):
    c = lax.axis_index("c")
    s = lax.axis_index("s")
    w = c * NS + s
    zero = jnp.zeros((16,), _f32)

    def zbody(i, _):
        rows2[0, i // 8, pl.ds((i % 8) * 16, 16)] = zero
        return 0

    lax.fori_loop(0, 64 * 8, zbody, 0)
    for m in range(SLICE // 64):
        pltpu.sync_copy(rows2.at[0], accS.at[pl.ds(s * SLICE + m * 64, 64)])

    plsc.subcore_barrier()

    pltpu.sync_copy(srcp_hbm.at[pl.ds(w * CH, CH)], idx_s)
    pltpu.sync_copy(dstp_hbm.at[pl.ds(w * CH, CH)], idx_d)

    def gather(j, h, b):
        pltpu.async_copy(table_hbm.at[idx_s.at[j, pl.ds(h * 64, 64)]],
                         rows2.at[b], sem)

    def drain(b):
        pltpu.make_async_copy(table_hbm.at[pl.ds(0, 64)], rows2.at[b],
                              sem).wait()

    def scatter(j, h, b):
        pltpu.sync_copy(rows2.at[b], accS.at[idx_d.at[j, h]], add=True)

    gather(0, 0, 0)

    def pbody(t, _):
        drain(0)
        gather(t, 1, 1)
        scatter(t, 0, 0)
        drain(1)

        @pl.when(t < CH - 1)
        def _g_next():
            gather(t + 1, 0, 0)

        scatter(t, 1, 1)
        return 0

    lax.fori_loop(0, CH, pbody, 0)

    plsc.subcore_barrier()
    pltpu.sync_copy(accS.at[pl.ds(s * SLICE, SLICE)],
                    p_hbm.at[pl.ds(c * NACC + s * SLICE, SLICE)])


@functools.partial(
    pl.kernel,
    out_type=jax.ShapeDtypeStruct((NC * NACC, D), _f32),
    mesh=_mesh(),
    compiler_params=pltpu.CompilerParams(needs_layout_passes=False),
    scratch_types=[
        pltpu.VMEM((CH, 128), _i32),
        pltpu.VMEM((CH, 2, 64), _i32),
        pltpu.VMEM((2, 64, D), _f32),
        pltpu.SemaphoreType.DMA,
        pltpu.VMEM_SHARED((NACC, D), _f32),
    ],
)
def _scatter_kernel(table_hbm, srcp_hbm, dstp_hbm, p_hbm,
                    idx_s, idx_d, rows2, sem, accS):
    _scatter_body(table_hbm, srcp_hbm, dstp_hbm, p_hbm,
                  idx_s, idx_d, rows2, sem, accS)


@functools.partial(
    pl.kernel,
    out_type=jax.ShapeDtypeStruct((BB, D), _f32),
    mesh=_mesh(),
    compiler_params=pltpu.CompilerParams(needs_layout_passes=False),
    scratch_types=[
        pltpu.VMEM((BB // NW, LL), _i32),
        pltpu.VMEM((BB // NW, D), _f32),
        pltpu.VMEM((2, 64, D), _f32),
        pltpu.SemaphoreType.DMA,
    ],
)
def _text_kernel(eids_hbm, ftab_hbm, tsum_hbm, tidv, tacc, rows2, sem):
    w = lax.axis_index("c") * NS + lax.axis_index("s")
    nseq = BB // NW
    pltpu.sync_copy(eids_hbm.at[pl.ds(w * nseq, nseq)], tidv)

    def tgather(j, h, b):
        pltpu.async_copy(ftab_hbm.at[tidv.at[j, pl.ds(h * 64, 64)]],
                         rows2.at[b], sem)

    def twait(b):
        pltpu.make_async_copy(ftab_hbm.at[pl.ds(0, 64)], rows2.at[b],
                              sem).wait()

    def taccum(b, carry):
        def rbody(i, cin):
            return tuple(cin[k] + rows2[b, i, pl.ds(k * 16, 16)]
                         for k in range(8))

        return lax.fori_loop(0, 64, rbody, carry)

    tgather(0, 0, 0)

    def tbody(j, _):
        twait(0)
        tgather(j, 1, 1)
        accs = taccum(0, tuple(jnp.zeros((16,), _f32) for _ in range(8)))
        twait(1)

        @pl.when(j < nseq - 1)
        def _t_next():
            tgather(j + 1, 0, 0)

        accs = taccum(1, accs)
        for k in range(8):
            tacc[j, pl.ds(k * 16, 16)] = accs[k]
        return 0

    lax.fori_loop(0, nseq, tbody, 0)
    pltpu.sync_copy(tacc, tsum_hbm.at[pl.ds(w * nseq, nseq)])


@functools.partial(
    pl.kernel,
    out_type=jax.ShapeDtypeStruct((BB, D), _f32),
    mesh=_mesh(),
    compiler_params=pltpu.CompilerParams(needs_layout_passes=False),
    scratch_types=[
        pltpu.VMEM((BB // NW,), _i32),
        pltpu.VMEM((BB // NW, D), _f32),
        pltpu.SemaphoreType.DMA,
    ],
)
def _gather_kernel(src_hbm, midx_hbm, out_hbm, idxv, rows, sem):
    w = lax.axis_index("c") * NS + lax.axis_index("s")
    n = BB // NW
    pltpu.sync_copy(midx_hbm.at[pl.ds(w * n, n)], idxv)
    pltpu.async_copy(src_hbm.at[idxv], rows, sem).wait()
    pltpu.sync_copy(rows, out_hbm.at[pl.ds(w * n, n)])


def _prep_body(deg_ref, x_ref, xs0_ref, dinv_ref):
    d = jnp.sum(deg_ref[...], axis=1) + 1.0
    dv = lax.rsqrt(d)
    xs0_ref[...] = x_ref[...] * dv[:, None]
    dinv_ref[...] = jnp.broadcast_to(dv[:, None], dinv_ref.shape)


def _prep_call(degp, x):
    blk = 1000
    return pl.pallas_call(
        _prep_body,
        grid=(NN // blk,),
        in_specs=[pl.BlockSpec((blk, NW), lambda j: (j, 0)),
                  pl.BlockSpec((blk, D), lambda j: (j, 0))],
        out_specs=[pl.BlockSpec((blk, D), lambda j: (j, 0)),
                   pl.BlockSpec((blk, D), lambda j: (j, 0))],
        out_shape=[jax.ShapeDtypeStruct((NN, D), _f32),
                   jax.ShapeDtypeStruct((NN, D), _f32)],
    )(degp, x)


def _mid_body(xs0_ref, p0_ref, p1_ref, dinv_ref, w1_ref, b1_ref, w2_ref,
              out_ref):
    dv = dinv_ref[...]
    agg = (xs0_ref[...] + p0_ref[...] + p1_ref[...]) * dv
    h = jnp.dot(agg, w1_ref[...], preferred_element_type=_f32) + b1_ref[...]
    h = jnp.where(h >= 0, h, 0.01 * h)
    z = jnp.dot(h, w2_ref[...], preferred_element_type=_f32)
    out_ref[...] = z * dv


def _mid_call(xs0, p0, p1, dinvb, W1, b1, W2):
    blk = 1000
    row = pl.BlockSpec((blk, D), lambda j: (j, 0))
    return pl.pallas_call(
        _mid_body,
        grid=(NN // blk,),
        in_specs=[row, row, row, row,
                  pl.BlockSpec((D, 2 * D), lambda j: (0, 0)),
                  pl.BlockSpec((1, 2 * D), lambda j: (0, 0)),
                  pl.BlockSpec((2 * D, D), lambda j: (0, 0))],
        out_specs=row,
        out_shape=jax.ShapeDtypeStruct((NN, D), _f32),
    )(xs0, p0, p1, dinvb, W1, b1, W2)


def _fin_body(xs2_ref, q0_ref, q1_ref, dinv_ref, b2_ref, out_ref):
    blk = out_ref.shape[0]
    j = pl.program_id(0)
    rows = j * blk + lax.broadcasted_iota(_i32, (blk, D), 0)
    v = (xs2_ref[...] + q0_ref[...] + q1_ref[...]) * dinv_ref[...] + b2_ref[...]
    out_ref[...] = jnp.where(rows < NN, v, 0.0)


def _fin_call(xs2, q0, q1, dinvb, b2):
    blk = 1112
    row = pl.BlockSpec((blk, D), lambda j: (j, 0))
    return pl.pallas_call(
        _fin_body,
        grid=(NPAD // blk,),
        in_specs=[row, row, row, row,
                  pl.BlockSpec((1, D), lambda j: (0, 0))],
        out_specs=row,
        out_shape=jax.ShapeDtypeStruct((NPAD, D), _f32),
    )(xs2, q0, q1, dinvb, b2)


def _textmm_body(ts_ref, wt_ref, bt_ref, out_ref):
    t = ts_ref[...] * (1.0 / LL)
    out_ref[...] = (jnp.dot(t, wt_ref[...], preferred_element_type=_f32)
                    + bt_ref[...])


def _textmm_call(tsum, W_text, b_text):
    return pl.pallas_call(
        _textmm_body,
        grid=(1,),
        in_specs=[pl.BlockSpec((BB, D), lambda j: (0, 0)),
                  pl.BlockSpec((D, D), lambda j: (0, 0)),
                  pl.BlockSpec((1, D), lambda j: (0, 0))],
        out_specs=pl.BlockSpec((BB, D), lambda j: (0, 0)),
        out_shape=jax.ShapeDtypeStruct((BB, D), _f32),
    )(tsum, W_text, b_text)


def kernel(e_ids, e_mask, x_graph, edge_index, batch_idx, data_mask,
           ft_table, W_text, b_text, W1, b1, W2, b2):
    x = x_graph.astype(_f32)
    src = edge_index[0].astype(_i32)
    dst = edge_index[1].astype(_i32)
    pad = EP - NE
    srcp = jnp.concatenate([src, jnp.zeros((pad,), _i32)]).reshape(NW * CH, 128)
    pad_dst = NN + (jnp.arange(pad, dtype=_i32) % (NACC - NN))
    dstf = jnp.concatenate([dst, pad_dst])
    dstp = dstf.reshape(NW * CH, 2, 64)

    degp = _deg_kernel(dstf)
    xs0, dinvb = _prep_call(degp.reshape(NW, DEGW).T, x)
    P = _scatter_kernel(xs0, srcp, dstp)
    tsum = _text_kernel(e_ids.astype(_i32), ft_table.astype(_f32))
    xs2 = _mid_call(xs0, P[:NN], P[NACC:NACC + NN], dinvb,
                    W1.astype(_f32), b1.reshape(1, -1).astype(_f32),
                    W2.astype(_f32))
    Q = _scatter_kernel(xs2, srcp, dstp)
    out2p = _fin_call(xs2, Q[:NN], Q[NACC:NACC + NN], dinvb,
                      b2.reshape(1, -1).astype(_f32))
    x_text = _textmm_call(tsum, W_text.astype(_f32),
                          b_text.reshape(1, -1).astype(_f32))
    midx = jnp.where(data_mask, batch_idx.astype(_i32), NN)
    out_graph = _gather_kernel(out2p, midx)
    return (x_text, out_graph)


# speedup vs baseline: 15.8801x; 1.7174x over previous
import functools

import jax
import jax.numpy as jnp
from jax import lax
from jax.experimental import pallas as pl
from jax.experimental.pallas import tpu as pltpu
from jax.experimental.pallas import tpu_sc as plsc

NN = 10000
NE = 320000
D = 128
BB = 1024
LL = 128
NC, NS = 2, 16
NW = NC * NS
CH = 80
EP = NW * CH * 128
NACC = 10240
SLICE = NACC // NS
NPAD = NN + 8
DEGW = NACC

_mesh = functools.partial(plsc.VectorSubcoreMesh,
                          core_axis_name="c", subcore_axis_name="s")

_f32 = jnp.float32
_i32 = jnp.int32


@functools.partial(
    pl.kernel,
    out_type=(jax.ShapeDtypeStruct((NW * DEGW,), _f32),
              jax.ShapeDtypeStruct((NACC,), _f32)),
    mesh=_mesh(),
    compiler_params=pltpu.CompilerParams(needs_layout_passes=False),
    scratch_types=[
        pltpu.VMEM((CH * 128,), _i32),
        pltpu.VMEM((DEGW,), _f32),
        pltpu.VMEM((BB,), _i32),
    ],
)
def _deg_kernel(dstf_hbm, bidx_hbm, out_hbm, bmu_hbm, dstbuf, acc, bidx):
    w = lax.axis_index("c") * NS + lax.axis_index("s")
    pltpu.sync_copy(dstf_hbm.at[pl.ds(w * CH * 128, CH * 128)], dstbuf)
    zero = jnp.zeros((16,), _f32)

    def zbody(i, _):
        acc[pl.ds(i * 16, 16)] = zero
        return 0

    lax.fori_loop(0, DEGW // 16, zbody, 0)
    ones = jnp.ones((16,), _f32)

    def body(i, _):
        idx = dstbuf[pl.ds(i * 16, 16)]
        plsc.addupdate_scatter(acc, [idx], ones)
        return 0

    lax.fori_loop(0, (CH * 128) // 16, body, 0)
    pltpu.sync_copy(acc, out_hbm.at[pl.ds(w * DEGW, DEGW)])

    @pl.when(w == 0)
    def _build_bitmap():
        def z2(i, _):
            acc[pl.ds(i * 16, 16)] = zero
            return 0

        lax.fori_loop(0, NACC // 16, z2, 0)
        pltpu.sync_copy(bidx_hbm, bidx)

        def sbody(i, _):
            idx = bidx[pl.ds(i * 16, 16)]
            plsc.store_scatter(acc, [idx], ones)
            return 0

        lax.fori_loop(0, BB // 16, sbody, 0)
        pltpu.sync_copy(acc.at[pl.ds(0, NACC)], bmu_hbm)


def _scatter_body(table_hbm, srcp_hbm, dstp_hbm, p_hbm,
                  idx_s, idx_d, rows2, sem, accS):
    c = lax.axis_index("c")
    s = lax.axis_index("s")
    w = c * NS + s
    zero = jnp.zeros((16,), _f32)

    def zbody(i, _):
        rows2[0, i // 8, pl.ds((i % 8) * 16, 16)] = zero
        return 0

    lax.fori_loop(0, 64 * 8, zbody, 0)
    for m in range(SLICE // 64):
        pltpu.sync_copy(rows2.at[0], accS.at[pl.ds(s * SLICE + m * 64, 64)])

    plsc.subcore_barrier()

    pltpu.sync_copy(srcp_hbm.at[pl.ds(w * CH, CH)], idx_s)
    pltpu.sync_copy(dstp_hbm.at[pl.ds(w * CH, CH)], idx_d)

    def gather(j, h, b):
        pltpu.async_copy(table_hbm.at[idx_s.at[j, pl.ds(h * 64, 64)]],
                         rows2.at[b], sem)

    def drain(b):
        pltpu.make_async_copy(table_hbm.at[pl.ds(0, 64)], rows2.at[b],
                              sem).wait()

    def scatter(j, h, b):
        pltpu.sync_copy(rows2.at[b], accS.at[idx_d.at[j, h]], add=True)

    gather(0, 0, 0)

    def pbody(t, _):
        drain(0)
        gather(t, 1, 1)
        scatter(t, 0, 0)
        drain(1)

        @pl.when(t < CH - 1)
        def _g_next():
            gather(t + 1, 0, 0)

        scatter(t, 1, 1)
        return 0

    lax.fori_loop(0, CH, pbody, 0)

    plsc.subcore_barrier()
    pltpu.sync_copy(accS.at[pl.ds(s * SLICE, SLICE)],
                    p_hbm.at[pl.ds(c * NACC + s * SLICE, SLICE)])


@functools.partial(
    pl.kernel,
    out_type=jax.ShapeDtypeStruct((NC * NACC, D), _f32),
    mesh=_mesh(),
    compiler_params=pltpu.CompilerParams(needs_layout_passes=False),
    scratch_types=[
        pltpu.VMEM((CH, 128), _i32),
        pltpu.VMEM((CH, 2, 64), _i32),
        pltpu.VMEM((2, 64, D), _f32),
        pltpu.SemaphoreType.DMA,
        pltpu.VMEM_SHARED((NACC, D), _f32),
    ],
)
def _scatter_kernel(table_hbm, srcp_hbm, dstp_hbm, p_hbm,
                    idx_s, idx_d, rows2, sem, accS):
    _scatter_body(table_hbm, srcp_hbm, dstp_hbm, p_hbm,
                  idx_s, idx_d, rows2, sem, accS)


@functools.partial(
    pl.kernel,
    out_type=jax.ShapeDtypeStruct((NC * NACC, D), _f32),
    mesh=_mesh(),
    compiler_params=pltpu.CompilerParams(needs_layout_passes=False),
    scratch_types=[
        pltpu.VMEM((CH, 128), _i32),
        pltpu.VMEM((CH * 128,), _i32),
        pltpu.VMEM((CH * 128 + 16,), _i32),
        pltpu.VMEM((CH * 128 + 16,), _i32),
        pltpu.VMEM((NACC // 32,), _i32),
        pltpu.VMEM((2, 16, D), _f32),
        pltpu.SemaphoreType.DMA,
        pltpu.VMEM_SHARED((NACC, D), _f32),
    ],
)
def _fscatter_kernel(table_hbm, srcp_hbm, dstf_hbm, bmp_hbm, p_hbm,
                     sraw, draw, sbuf, dbuf, bmp, rows2, sem, accS):
    c = lax.axis_index("c")
    s = lax.axis_index("s")
    w = c * NS + s
    zero = jnp.zeros((16,), _f32)
    lanes = lax.iota(_i32, 16)

    def zrow(i, _):
        rows2[0, i // 8, pl.ds((i % 8) * 16, 16)] = zero
        return 0

    lax.fori_loop(0, 16 * 8, zrow, 0)
    for m in range(SLICE // 16):
        pltpu.sync_copy(rows2.at[0], accS.at[pl.ds(s * SLICE + m * 16, 16)])

    plsc.subcore_barrier()

    pltpu.sync_copy(srcp_hbm.at[pl.ds(w * CH, CH)], sraw)
    pltpu.sync_copy(dstf_hbm.at[pl.ds(w * CH * 128, CH * 128)], draw)
    pltpu.sync_copy(bmp_hbm, bmp)

    dummy_dst = NN + lanes
    zero_i = jnp.zeros((16,), _i32)

    def pfill(i, _):
        sbuf[pl.ds(i * 16, 16)] = zero_i
        dbuf[pl.ds(i * 16, 16)] = dummy_dst
        return 0

    lax.fori_loop(0, (CH * 128 + 16) // 16, pfill, 0)

    def fbody(i, off):
        svec = sraw[i // 8, pl.ds((i % 8) * 16, 16)]
        dvec = draw[pl.ds(i * 16, 16)]
        word = plsc.load_gather(bmp, [lax.shift_right_logical(dvec, 5)])
        bit = lax.shift_right_logical(word, dvec & 31) & 1
        msk = bit != 0
        store_window_s = sbuf.at[pl.ds(off, 16)]
        store_window_d = dbuf.at[pl.ds(off, 16)]
        plsc.store_compressed(store_window_s, svec, mask=msk)
        plsc.store_compressed(store_window_d, dvec, mask=msk)
        cnt = plsc.all_reduce_population_count(msk)
        return off + cnt[0]

    off = lax.fori_loop(0, (CH * 128) // 16, fbody, jnp.int32(0))
    nv = lax.max((off + 15) // 16, 1)

    def gather(t, b):
        svec = sbuf[pl.ds(t * 16, 16)]
        pltpu.async_copy(table_hbm.at[svec], rows2.at[b], sem)

    def drain(b):
        pltpu.make_async_copy(table_hbm.at[pl.ds(0, 16)], rows2.at[b],
                              sem).wait()

    gather(0, 0)

    def pbody(t, _):
        b = t % 2
        drain(b)

        @pl.when(t + 1 < nv)
        def _g_next():
            gather(t + 1, 1 - b)

        dvec = dbuf[pl.ds(t * 16, 16)]
        pltpu.sync_copy(rows2.at[b], accS.at[dvec], add=True)
        return 0

    lax.fori_loop(0, nv, pbody, 0)

    plsc.subcore_barrier()
    pltpu.sync_copy(accS.at[pl.ds(s * SLICE, SLICE)],
                    p_hbm.at[pl.ds(c * NACC + s * SLICE, SLICE)])


@functools.partial(
    pl.kernel,
    out_type=jax.ShapeDtypeStruct((BB, D), _f32),
    mesh=_mesh(),
    compiler_params=pltpu.CompilerParams(needs_layout_passes=False),
    scratch_types=[
        pltpu.VMEM((BB // NW, LL), _i32),
        pltpu.VMEM((BB // NW, D), _f32),
        pltpu.VMEM((2, 64, D), _f32),
        pltpu.SemaphoreType.DMA,
    ],
)
def _text_kernel(eids_hbm, ftab_hbm, tsum_hbm, tidv, tacc, rows2, sem):
    w = lax.axis_index("c") * NS + lax.axis_index("s")
    nseq = BB // NW
    pltpu.sync_copy(eids_hbm.at[pl.ds(w * nseq, nseq)], tidv)

    def tgather(j, h, b):
        pltpu.async_copy(ftab_hbm.at[tidv.at[j, pl.ds(h * 64, 64)]],
                         rows2.at[b], sem)

    def twait(b):
        pltpu.make_async_copy(ftab_hbm.at[pl.ds(0, 64)], rows2.at[b],
                              sem).wait()

    def taccum(b, carry):
        def rbody(i, cin):
            return tuple(cin[k] + rows2[b, i, pl.ds(k * 16, 16)]
                         for k in range(8))

        return lax.fori_loop(0, 64, rbody, carry)

    tgather(0, 0, 0)

    def tbody(j, _):
        twait(0)
        tgather(j, 1, 1)
        accs = taccum(0, tuple(jnp.zeros((16,), _f32) for _ in range(8)))
        twait(1)

        @pl.when(j < nseq - 1)
        def _t_next():
            tgather(j + 1, 0, 0)

        accs = taccum(1, accs)
        for k in range(8):
            tacc[j, pl.ds(k * 16, 16)] = accs[k]
        return 0

    lax.fori_loop(0, nseq, tbody, 0)
    pltpu.sync_copy(tacc, tsum_hbm.at[pl.ds(w * nseq, nseq)])


@functools.partial(
    pl.kernel,
    out_type=jax.ShapeDtypeStruct((BB, D), _f32),
    mesh=_mesh(),
    compiler_params=pltpu.CompilerParams(needs_layout_passes=False),
    scratch_types=[
        pltpu.VMEM((BB // NW,), _i32),
        pltpu.VMEM((BB // NW, D), _f32),
        pltpu.SemaphoreType.DMA,
    ],
)
def _gather_kernel(src_hbm, midx_hbm, out_hbm, idxv, rows, sem):
    w = lax.axis_index("c") * NS + lax.axis_index("s")
    n = BB // NW
    pltpu.sync_copy(midx_hbm.at[pl.ds(w * n, n)], idxv)
    pltpu.async_copy(src_hbm.at[idxv], rows, sem).wait()
    pltpu.sync_copy(rows, out_hbm.at[pl.ds(w * n, n)])


def _prep_body(deg_ref, x_ref, xs0_ref, dinv_ref):
    d = jnp.sum(deg_ref[...], axis=1) + 1.0
    dv = lax.rsqrt(d)
    xs0_ref[...] = x_ref[...] * dv[:, None]
    dinv_ref[...] = jnp.broadcast_to(dv[:, None], dinv_ref.shape)


def _prep_call(degp, x):
    blk = 1000
    return pl.pallas_call(
        _prep_body,
        grid=(NN // blk,),
        in_specs=[pl.BlockSpec((blk, NW), lambda j: (j, 0)),
                  pl.BlockSpec((blk, D), lambda j: (j, 0))],
        out_specs=[pl.BlockSpec((blk, D), lambda j: (j, 0)),
                   pl.BlockSpec((blk, D), lambda j: (j, 0))],
        out_shape=[jax.ShapeDtypeStruct((NN, D), _f32),
                   jax.ShapeDtypeStruct((NN, D), _f32)],
    )(degp, x)


def _mid_body(xs0_ref, p0_ref, p1_ref, dinv_ref, w1_ref, b1_ref, w2_ref,
              out_ref):
    dv = dinv_ref[...]
    agg = (xs0_ref[...] + p0_ref[...] + p1_ref[...]) * dv
    h = jnp.dot(agg, w1_ref[...], preferred_element_type=_f32) + b1_ref[...]
    h = jnp.where(h >= 0, h, 0.01 * h)
    z = jnp.dot(h, w2_ref[...], preferred_element_type=_f32)
    out_ref[...] = z * dv


def _mid_call(xs0, p0, p1, dinvb, W1, b1, W2):
    blk = 1000
    row = pl.BlockSpec((blk, D), lambda j: (j, 0))
    return pl.pallas_call(
        _mid_body,
        grid=(NN // blk,),
        in_specs=[row, row, row, row,
                  pl.BlockSpec((D, 2 * D), lambda j: (0, 0)),
                  pl.BlockSpec((1, 2 * D), lambda j: (0, 0)),
                  pl.BlockSpec((2 * D, D), lambda j: (0, 0))],
        out_specs=row,
        out_shape=jax.ShapeDtypeStruct((NN, D), _f32),
    )(xs0, p0, p1, dinvb, W1, b1, W2)


def _fin_body(xs2_ref, q0_ref, q1_ref, dinv_ref, b2_ref, out_ref):
    blk = out_ref.shape[0]
    j = pl.program_id(0)
    rows = j * blk + lax.broadcasted_iota(_i32, (blk, D), 0)
    v = (xs2_ref[...] + q0_ref[...] + q1_ref[...]) * dinv_ref[...] + b2_ref[...]
    out_ref[...] = jnp.where(rows < NN, v, 0.0)


def _fin_call(xs2, q0, q1, dinvb, b2):
    blk = 1112
    row = pl.BlockSpec((blk, D), lambda j: (j, 0))
    return pl.pallas_call(
        _fin_body,
        grid=(NPAD // blk,),
        in_specs=[row, row, row, row,
                  pl.BlockSpec((1, D), lambda j: (0, 0))],
        out_specs=row,
        out_shape=jax.ShapeDtypeStruct((NPAD, D), _f32),
    )(xs2, q0, q1, dinvb, b2)


def _textmm_body(ts_ref, wt_ref, bt_ref, out_ref):
    t = ts_ref[...] * (1.0 / LL)
    out_ref[...] = (jnp.dot(t, wt_ref[...], preferred_element_type=_f32)
                    + bt_ref[...])


def _textmm_call(tsum, W_text, b_text):
    return pl.pallas_call(
        _textmm_body,
        grid=(1,),
        in_specs=[pl.BlockSpec((BB, D), lambda j: (0, 0)),
                  pl.BlockSpec((D, D), lambda j: (0, 0)),
                  pl.BlockSpec((1, D), lambda j: (0, 0))],
        out_specs=pl.BlockSpec((BB, D), lambda j: (0, 0)),
        out_shape=jax.ShapeDtypeStruct((BB, D), _f32),
    )(tsum, W_text, b_text)


def _bmpack_body(bm_ref, out_ref):
    bits = (bm_ref[...] > 0).astype(_i32)
    shifted = jnp.left_shift(bits, lax.broadcasted_iota(_i32, bits.shape, 1))
    out_ref[...] = jnp.sum(shifted, axis=1, keepdims=True)


def _bmpack_call(bmu):
    n = NACC // 32
    return pl.pallas_call(
        _bmpack_body,
        grid=(1,),
        in_specs=[pl.BlockSpec((n, 32), lambda j: (0, 0))],
        out_specs=pl.BlockSpec((n, 1), lambda j: (0, 0)),
        out_shape=jax.ShapeDtypeStruct((n, 1), _i32),
    )(bmu.reshape(n, 32))


def kernel(e_ids, e_mask, x_graph, edge_index, batch_idx, data_mask,
           ft_table, W_text, b_text, W1, b1, W2, b2):
    x = x_graph.astype(_f32)
    src = edge_index[0].astype(_i32)
    dst = edge_index[1].astype(_i32)
    per_w = NE // NW
    pad_w = EP // NW - per_w
    src_w = jnp.concatenate(
        [src.reshape(NW, per_w), jnp.zeros((NW, pad_w), _i32)], axis=1)
    pad_dst = jnp.broadcast_to(NN + jnp.arange(pad_w, dtype=_i32),
                               (NW, pad_w))
    dst_w = jnp.concatenate([dst.reshape(NW, per_w), pad_dst], axis=1)
    srcp = src_w.reshape(NW * CH, 128)
    dstf = dst_w.reshape(-1)
    dstp = dstf.reshape(NW * CH, 2, 64)

    bidx32 = batch_idx.astype(_i32)
    degp, bmu = _deg_kernel(dstf, bidx32)
    bmp = _bmpack_call(bmu).reshape(-1)
    xs0, dinvb = _prep_call(degp.reshape(NW, DEGW).T, x)
    P = _scatter_kernel(xs0, srcp, dstp)
    tsum = _text_kernel(e_ids.astype(_i32), ft_table.astype(_f32))
    xs2 = _mid_call(xs0, P[:NN], P[NACC:NACC + NN], dinvb,
                    W1.astype(_f32), b1.reshape(1, -1).astype(_f32),
                    W2.astype(_f32))
    Q = _fscatter_kernel(xs2, srcp, dstf, bmp)
    out2p = _fin_call(xs2, Q[:NN], Q[NACC:NACC + NN], dinvb,
                      b2.reshape(1, -1).astype(_f32))
    x_text = _textmm_call(tsum, W_text.astype(_f32),
                          b_text.reshape(1, -1).astype(_f32))
    midx = jnp.where(data_mask, bidx32, NN)
    out_graph = _gather_kernel(out2p, midx)
    return (x_text, out_graph)


# speedup vs baseline: 17.4123x; 1.0965x over previous
import functools

import jax
import jax.numpy as jnp
from jax import lax
from jax.experimental import pallas as pl
from jax.experimental.pallas import tpu as pltpu
from jax.experimental.pallas import tpu_sc as plsc

NN = 10000
NE = 320000
D = 128
BB = 1024
LL = 128
NC, NS = 2, 16
NW = NC * NS
CH = 80
EP = NW * CH * 128
NACC = 10240
SLICE = NACC // NS
NPAD = NN + 8
DEGW = NACC

_mesh = functools.partial(plsc.VectorSubcoreMesh,
                          core_axis_name="c", subcore_axis_name="s")

_f32 = jnp.float32
_i32 = jnp.int32


@functools.partial(
    pl.kernel,
    out_type=(jax.ShapeDtypeStruct((NW * DEGW,), _f32),
              jax.ShapeDtypeStruct((NACC,), _f32)),
    mesh=_mesh(),
    compiler_params=pltpu.CompilerParams(needs_layout_passes=False),
    scratch_types=[
        pltpu.VMEM((CH * 128,), _i32),
        pltpu.VMEM((DEGW,), _f32),
        pltpu.VMEM((BB,), _i32),
    ],
)
def _deg_kernel(dstf_hbm, bidx_hbm, out_hbm, bmu_hbm, dstbuf, acc, bidx):
    w = lax.axis_index("c") * NS + lax.axis_index("s")
    pltpu.sync_copy(dstf_hbm.at[pl.ds(w * CH * 128, CH * 128)], dstbuf)
    zero = jnp.zeros((16,), _f32)

    def zbody(i, _):
        acc[pl.ds(i * 16, 16)] = zero
        return 0

    lax.fori_loop(0, DEGW // 16, zbody, 0)
    ones = jnp.ones((16,), _f32)

    def body(i, _):
        idx = dstbuf[pl.ds(i * 16, 16)]
        plsc.addupdate_scatter(acc, [idx], ones)
        return 0

    lax.fori_loop(0, (CH * 128) // 16, body, 0)
    pltpu.sync_copy(acc, out_hbm.at[pl.ds(w * DEGW, DEGW)])

    @pl.when(w == 0)
    def _build_bitmap():
        def z2(i, _):
            acc[pl.ds(i * 16, 16)] = zero
            return 0

        lax.fori_loop(0, NACC // 16, z2, 0)
        pltpu.sync_copy(bidx_hbm, bidx)

        def sbody(i, _):
            idx = bidx[pl.ds(i * 16, 16)]
            plsc.store_scatter(acc, [idx], ones)
            return 0

        lax.fori_loop(0, BB // 16, sbody, 0)
        pltpu.sync_copy(acc.at[pl.ds(0, NACC)], bmu_hbm)


def _scatter_body(table_hbm, srcp_hbm, dstp_hbm, p_hbm,
                  idx_s, idx_d, rows3, sem, accS):
    c = lax.axis_index("c")
    s = lax.axis_index("s")
    w = c * NS + s
    zero = jnp.zeros((16,), _f32)

    def zbody(i, _):
        rows3[0, i // 8, pl.ds((i % 8) * 16, 16)] = zero
        return 0

    lax.fori_loop(0, 64 * 8, zbody, 0)
    for m in range(SLICE // 64):
        pltpu.sync_copy(rows3.at[0], accS.at[pl.ds(s * SLICE + m * 64, 64)])

    plsc.subcore_barrier()

    pltpu.sync_copy(srcp_hbm.at[pl.ds(w * CH, CH)], idx_s)
    pltpu.sync_copy(dstp_hbm.at[pl.ds(w * CH, CH)], idx_d)

    NT = CH * 2

    def gather(t, b):
        pltpu.async_copy(
            table_hbm.at[idx_s.at[t // 2, pl.ds((t % 2) * 64, 64)]],
            rows3.at[b], sem)

    def drain(b):
        pltpu.make_async_copy(table_hbm.at[pl.ds(0, 64)], rows3.at[b],
                              sem).wait()

    def scatter(t, b):
        for q in range(4):
            dvec = idx_d[t // 2, pl.ds((t % 2) * 64 + q * 16, 16)]
            pltpu.sync_copy(rows3.at[b].at[pl.ds(q * 16, 16)],
                            accS.at[dvec], add=True)

    gather(0, 0)
    gather(1, 1)

    def pbody(t, _):
        b = t % 3
        drain(b)

        @pl.when(t + 2 < NT)
        def _g_next():
            gather(t + 2, (t + 2) % 3)

        scatter(t, b)
        return 0

    lax.fori_loop(0, NT, pbody, 0)

    plsc.subcore_barrier()
    pltpu.sync_copy(accS.at[pl.ds(s * SLICE, SLICE)],
                    p_hbm.at[pl.ds(c * NACC + s * SLICE, SLICE)])


@functools.partial(
    pl.kernel,
    out_type=jax.ShapeDtypeStruct((NC * NACC, D), _f32),
    mesh=_mesh(),
    compiler_params=pltpu.CompilerParams(needs_layout_passes=False),
    scratch_types=[
        pltpu.VMEM((CH, 128), _i32),
        pltpu.VMEM((CH, 128), _i32),
        pltpu.VMEM((3, 64, D), _f32),
        pltpu.SemaphoreType.DMA,
        pltpu.VMEM_SHARED((NACC, D), _f32),
    ],
)
def _scatter_kernel(table_hbm, srcp_hbm, dstp_hbm, p_hbm,
                    idx_s, idx_d, rows3, sem, accS):
    _scatter_body(table_hbm, srcp_hbm, dstp_hbm, p_hbm,
                  idx_s, idx_d, rows3, sem, accS)


@functools.partial(
    pl.kernel,
    out_type=jax.ShapeDtypeStruct((NC * NACC, D), _f32),
    mesh=_mesh(),
    compiler_params=pltpu.CompilerParams(needs_layout_passes=False),
    scratch_types=[
        pltpu.VMEM((CH, 128), _i32),
        pltpu.VMEM((CH * 128,), _i32),
        pltpu.VMEM((CH * 128 + 16,), _i32),
        pltpu.VMEM((CH * 128 + 16,), _i32),
        pltpu.VMEM((NACC // 32,), _i32),
        pltpu.VMEM((2, 16, D), _f32),
        pltpu.SemaphoreType.DMA,
        pltpu.VMEM_SHARED((NACC, D), _f32),
    ],
)
def _fscatter_kernel(table_hbm, srcp_hbm, dstf_hbm, bmp_hbm, p_hbm,
                     sraw, draw, sbuf, dbuf, bmp, rows2, sem, accS):
    c = lax.axis_index("c")
    s = lax.axis_index("s")
    w = c * NS + s
    zero = jnp.zeros((16,), _f32)
    lanes = lax.iota(_i32, 16)

    def zrow(i, _):
        rows2[0, i // 8, pl.ds((i % 8) * 16, 16)] = zero
        return 0

    lax.fori_loop(0, 16 * 8, zrow, 0)
    for m in range(SLICE // 16):
        pltpu.sync_copy(rows2.at[0], accS.at[pl.ds(s * SLICE + m * 16, 16)])

    plsc.subcore_barrier()

    pltpu.sync_copy(srcp_hbm.at[pl.ds(w * CH, CH)], sraw)
    pltpu.sync_copy(dstf_hbm.at[pl.ds(w * CH * 128, CH * 128)], draw)
    pltpu.sync_copy(bmp_hbm, bmp)

    dummy_dst = NN + lanes
    zero_i = jnp.zeros((16,), _i32)

    def pfill(i, _):
        sbuf[pl.ds(i * 16, 16)] = zero_i
        dbuf[pl.ds(i * 16, 16)] = dummy_dst
        return 0

    lax.fori_loop(0, (CH * 128 + 16) // 16, pfill, 0)

    def fbody(i, off):
        svec = sraw[i // 8, pl.ds((i % 8) * 16, 16)]
        dvec = draw[pl.ds(i * 16, 16)]
        word = plsc.load_gather(bmp, [lax.shift_right_logical(dvec, 5)])
        bit = lax.shift_right_logical(word, dvec & 31) & 1
        msk = bit != 0
        store_window_s = sbuf.at[pl.ds(off, 16)]
        store_window_d = dbuf.at[pl.ds(off, 16)]
        plsc.store_compressed(store_window_s, svec, mask=msk)
        plsc.store_compressed(store_window_d, dvec, mask=msk)
        cnt = plsc.all_reduce_population_count(msk)
        return off + cnt[0]

    off = lax.fori_loop(0, (CH * 128) // 16, fbody, jnp.int32(0))
    nv = lax.max((off + 15) // 16, 1)

    def gather(t, b):
        svec = sbuf[pl.ds(t * 16, 16)]
        pltpu.async_copy(table_hbm.at[svec], rows2.at[b], sem)

    def drain(b):
        pltpu.make_async_copy(table_hbm.at[pl.ds(0, 16)], rows2.at[b],
                              sem).wait()

    gather(0, 0)

    def pbody(t, _):
        b = t % 2
        drain(b)

        @pl.when(t + 1 < nv)
        def _g_next():
            gather(t + 1, 1 - b)

        dvec = dbuf[pl.ds(t * 16, 16)]
        pltpu.sync_copy(rows2.at[b], accS.at[dvec], add=True)
        return 0

    lax.fori_loop(0, nv, pbody, 0)

    plsc.subcore_barrier()
    pltpu.sync_copy(accS.at[pl.ds(s * SLICE, SLICE)],
                    p_hbm.at[pl.ds(c * NACC + s * SLICE, SLICE)])


@functools.partial(
    pl.kernel,
    out_type=jax.ShapeDtypeStruct((BB, D), _f32),
    mesh=_mesh(),
    compiler_params=pltpu.CompilerParams(needs_layout_passes=False),
    scratch_types=[
        pltpu.VMEM((BB // NW, LL), _i32),
        pltpu.VMEM((BB // NW, D), _f32),
        pltpu.VMEM((2, 64, D), _f32),
        pltpu.SemaphoreType.DMA,
    ],
)
def _text_kernel(eids_hbm, ftab_hbm, tsum_hbm, tidv, tacc, rows2, sem):
    w = lax.axis_index("c") * NS + lax.axis_index("s")
    nseq = BB // NW
    pltpu.sync_copy(eids_hbm.at[pl.ds(w * nseq, nseq)], tidv)

    def tgather(j, h, b):
        pltpu.async_copy(ftab_hbm.at[tidv.at[j, pl.ds(h * 64, 64)]],
                         rows2.at[b], sem)

    def twait(b):
        pltpu.make_async_copy(ftab_hbm.at[pl.ds(0, 64)], rows2.at[b],
                              sem).wait()

    def taccum(b, carry):
        def rbody(i, cin):
            return tuple(cin[k] + rows2[b, i, pl.ds(k * 16, 16)]
                         for k in range(8))

        return lax.fori_loop(0, 64, rbody, carry)

    tgather(0, 0, 0)

    def tbody(j, _):
        twait(0)
        tgather(j, 1, 1)
        accs = taccum(0, tuple(jnp.zeros((16,), _f32) for _ in range(8)))
        twait(1)

        @pl.when(j < nseq - 1)
        def _t_next():
            tgather(j + 1, 0, 0)

        accs = taccum(1, accs)
        for k in range(8):
            tacc[j, pl.ds(k * 16, 16)] = accs[k]
        return 0

    lax.fori_loop(0, nseq, tbody, 0)
    pltpu.sync_copy(tacc, tsum_hbm.at[pl.ds(w * nseq, nseq)])


@functools.partial(
    pl.kernel,
    out_type=jax.ShapeDtypeStruct((BB, D), _f32),
    mesh=_mesh(),
    compiler_params=pltpu.CompilerParams(needs_layout_passes=False),
    scratch_types=[
        pltpu.VMEM((BB // NW,), _i32),
        pltpu.VMEM((BB // NW, D), _f32),
        pltpu.SemaphoreType.DMA,
    ],
)
def _gather_kernel(src_hbm, midx_hbm, out_hbm, idxv, rows, sem):
    w = lax.axis_index("c") * NS + lax.axis_index("s")
    n = BB // NW
    pltpu.sync_copy(midx_hbm.at[pl.ds(w * n, n)], idxv)
    pltpu.async_copy(src_hbm.at[idxv], rows, sem).wait()
    pltpu.sync_copy(rows, out_hbm.at[pl.ds(w * n, n)])


def _prep_body(deg_ref, x_ref, xs0_ref, dinv_ref):
    d = jnp.sum(deg_ref[...], axis=1) + 1.0
    dv = lax.rsqrt(d)
    xs0_ref[...] = x_ref[...] * dv[:, None]
    dinv_ref[...] = jnp.broadcast_to(dv[:, None], dinv_ref.shape)


def _prep_call(degp, x):
    blk = 1000
    return pl.pallas_call(
        _prep_body,
        grid=(NN // blk,),
        in_specs=[pl.BlockSpec((blk, NW), lambda j: (j, 0)),
                  pl.BlockSpec((blk, D), lambda j: (j, 0))],
        out_specs=[pl.BlockSpec((blk, D), lambda j: (j, 0)),
                   pl.BlockSpec((blk, D), lambda j: (j, 0))],
        out_shape=[jax.ShapeDtypeStruct((NN, D), _f32),
                   jax.ShapeDtypeStruct((NN, D), _f32)],
    )(degp, x)


def _mid_body(xs0_ref, p0_ref, p1_ref, dinv_ref, w1_ref, b1_ref, w2_ref,
              out_ref):
    dv = dinv_ref[...]
    agg = (xs0_ref[...] + p0_ref[...] + p1_ref[...]) * dv
    h = jnp.dot(agg, w1_ref[...], preferred_element_type=_f32) + b1_ref[...]
    h = jnp.where(h >= 0, h, 0.01 * h)
    z = jnp.dot(h, w2_ref[...], preferred_element_type=_f32)
    out_ref[...] = z * dv


def _mid_call(xs0, p0, p1, dinvb, W1, b1, W2):
    blk = 1000
    row = pl.BlockSpec((blk, D), lambda j: (j, 0))
    return pl.pallas_call(
        _mid_body,
        grid=(NN // blk,),
        in_specs=[row, row, row, row,
                  pl.BlockSpec((D, 2 * D), lambda j: (0, 0)),
                  pl.BlockSpec((1, 2 * D), lambda j: (0, 0)),
                  pl.BlockSpec((2 * D, D), lambda j: (0, 0))],
        out_specs=row,
        out_shape=jax.ShapeDtypeStruct((NN, D), _f32),
    )(xs0, p0, p1, dinvb, W1, b1, W2)


def _fin_body(xs2_ref, q0_ref, q1_ref, dinv_ref, b2_ref, out_ref):
    blk = out_ref.shape[0]
    j = pl.program_id(0)
    rows = j * blk + lax.broadcasted_iota(_i32, (blk, D), 0)
    v = (xs2_ref[...] + q0_ref[...] + q1_ref[...]) * dinv_ref[...] + b2_ref[...]
    out_ref[...] = jnp.where(rows < NN, v, 0.0)


def _fin_call(xs2, q0, q1, dinvb, b2):
    blk = 1112
    row = pl.BlockSpec((blk, D), lambda j: (j, 0))
    return pl.pallas_call(
        _fin_body,
        grid=(NPAD // blk,),
        in_specs=[row, row, row, row,
                  pl.BlockSpec((1, D), lambda j: (0, 0))],
        out_specs=row,
        out_shape=jax.ShapeDtypeStruct((NPAD, D), _f32),
    )(xs2, q0, q1, dinvb, b2)


def _textmm_body(ts_ref, wt_ref, bt_ref, out_ref):
    t = ts_ref[...] * (1.0 / LL)
    out_ref[...] = (jnp.dot(t, wt_ref[...], preferred_element_type=_f32)
                    + bt_ref[...])


def _textmm_call(tsum, W_text, b_text):
    return pl.pallas_call(
        _textmm_body,
        grid=(1,),
        in_specs=[pl.BlockSpec((BB, D), lambda j: (0, 0)),
                  pl.BlockSpec((D, D), lambda j: (0, 0)),
                  pl.BlockSpec((1, D), lambda j: (0, 0))],
        out_specs=pl.BlockSpec((BB, D), lambda j: (0, 0)),
        out_shape=jax.ShapeDtypeStruct((BB, D), _f32),
    )(tsum, W_text, b_text)


def _bmpack_body(bm_ref, out_ref):
    bits = (bm_ref[...] > 0).astype(_i32)
    shifted = jnp.left_shift(bits, lax.broadcasted_iota(_i32, bits.shape, 1))
    out_ref[...] = jnp.sum(shifted, axis=1, keepdims=True)


def _bmpack_call(bmu):
    n = NACC // 32
    return pl.pallas_call(
        _bmpack_body,
        grid=(1,),
        in_specs=[pl.BlockSpec((n, 32), lambda j: (0, 0))],
        out_specs=pl.BlockSpec((n, 1), lambda j: (0, 0)),
        out_shape=jax.ShapeDtypeStruct((n, 1), _i32),
    )(bmu.reshape(n, 32))


def kernel(e_ids, e_mask, x_graph, edge_index, batch_idx, data_mask,
           ft_table, W_text, b_text, W1, b1, W2, b2):
    x = x_graph.astype(_f32)
    src = edge_index[0].astype(_i32)
    dst = edge_index[1].astype(_i32)
    per_w = NE // NW
    pad_w = EP // NW - per_w
    src_w = jnp.concatenate(
        [src.reshape(NW, per_w), jnp.zeros((NW, pad_w), _i32)], axis=1)
    pad_dst = jnp.broadcast_to(NN + jnp.arange(pad_w, dtype=_i32),
                               (NW, pad_w))
    dst_w = jnp.concatenate([dst.reshape(NW, per_w), pad_dst], axis=1)
    srcp = src_w.reshape(NW * CH, 128)
    dstf = dst_w.reshape(-1)
    dstp = dstf.reshape(NW * CH, 128)

    bidx32 = batch_idx.astype(_i32)
    degp, bmu = _deg_kernel(dstf, bidx32)
    bmp = _bmpack_call(bmu).reshape(-1)
    xs0, dinvb = _prep_call(degp.reshape(NW, DEGW).T, x)
    P = _scatter_kernel(xs0, srcp, dstp)
    tsum = _text_kernel(e_ids.astype(_i32), ft_table.astype(_f32))
    xs2 = _mid_call(xs0, P[:NN], P[NACC:NACC + NN], dinvb,
                    W1.astype(_f32), b1.reshape(1, -1).astype(_f32),
                    W2.astype(_f32))
    Q = _fscatter_kernel(xs2, srcp, dstf, bmp)
    out2p = _fin_call(xs2, Q[:NN], Q[NACC:NACC + NN], dinvb,
                      b2.reshape(1, -1).astype(_f32))
    x_text = _textmm_call(tsum, W_text.astype(_f32),
                          b_text.reshape(1, -1).astype(_f32))
    midx = jnp.where(data_mask, bidx32, NN)
    out_graph = _gather_kernel(out2p, midx)
    return (x_text, out_graph)


# speedup vs baseline: 17.4787x; 1.0038x over previous
import functools

import jax
import jax.numpy as jnp
from jax import lax
from jax.experimental import pallas as pl
from jax.experimental.pallas import tpu as pltpu
from jax.experimental.pallas import tpu_sc as plsc

NN = 10000
NE = 320000
D = 128
BB = 1024
LL = 128
NC, NS = 2, 16
NW = NC * NS
CH = 80
EP = NW * CH * 128
NACC = 10240
SLICE = NACC // NS
NPAD = NN + 8
DEGW = NACC

_mesh = functools.partial(plsc.VectorSubcoreMesh,
                          core_axis_name="c", subcore_axis_name="s")

_f32 = jnp.float32
_i32 = jnp.int32


@functools.partial(
    pl.kernel,
    out_type=(jax.ShapeDtypeStruct((NW * DEGW,), _f32),
              jax.ShapeDtypeStruct((NACC,), _f32)),
    mesh=_mesh(),
    compiler_params=pltpu.CompilerParams(needs_layout_passes=False),
    scratch_types=[
        pltpu.VMEM((CH * 128,), _i32),
        pltpu.VMEM((DEGW,), _f32),
        pltpu.VMEM((BB,), _i32),
    ],
)
def _deg_kernel(dstf_hbm, bidx_hbm, out_hbm, bmu_hbm, dstbuf, acc, bidx):
    w = lax.axis_index("c") * NS + lax.axis_index("s")
    pltpu.sync_copy(dstf_hbm.at[pl.ds(w * CH * 128, CH * 128)], dstbuf)
    zero = jnp.zeros((16,), _f32)

    def zbody(i, _):
        acc[pl.ds(i * 16, 16)] = zero
        return 0

    lax.fori_loop(0, DEGW // 16, zbody, 0)
    ones = jnp.ones((16,), _f32)

    def body(i, _):
        idx = dstbuf[pl.ds(i * 16, 16)]
        plsc.addupdate_scatter(acc, [idx], ones)
        return 0

    lax.fori_loop(0, (CH * 128) // 16, body, 0)
    pltpu.sync_copy(acc, out_hbm.at[pl.ds(w * DEGW, DEGW)])

    @pl.when(w == 0)
    def _build_bitmap():
        def z2(i, _):
            acc[pl.ds(i * 16, 16)] = zero
            return 0

        lax.fori_loop(0, NACC // 16, z2, 0)
        pltpu.sync_copy(bidx_hbm, bidx)

        def sbody(i, _):
            idx = bidx[pl.ds(i * 16, 16)]
            plsc.store_scatter(acc, [idx], ones)
            return 0

        lax.fori_loop(0, BB // 16, sbody, 0)
        pltpu.sync_copy(acc.at[pl.ds(0, NACC)], bmu_hbm)


def _scatter_body(table_hbm, srcp_hbm, dstp_hbm, p_hbm,
                  idx_s, idx_d, rows3, sem, accS):
    c = lax.axis_index("c")
    s = lax.axis_index("s")
    w = c * NS + s
    zero = jnp.zeros((16,), _f32)

    def zbody(i, _):
        rows3[0, i // 8, pl.ds((i % 8) * 16, 16)] = zero
        return 0

    lax.fori_loop(0, 32 * 8, zbody, 0)
    for m in range(SLICE // 32):
        pltpu.sync_copy(rows3.at[0], accS.at[pl.ds(s * SLICE + m * 32, 32)])

    plsc.subcore_barrier()

    pltpu.sync_copy(srcp_hbm.at[pl.ds(w * CH, CH)], idx_s)
    pltpu.sync_copy(dstp_hbm.at[pl.ds(w * CH, CH)], idx_d)

    NT = CH * 4

    def gather(t, b):
        pltpu.async_copy(
            table_hbm.at[idx_s.at[t // 4, pl.ds((t % 4) * 32, 32)]],
            rows3.at[b], sem)

    def drain(b):
        pltpu.make_async_copy(table_hbm.at[pl.ds(0, 32)], rows3.at[b],
                              sem).wait()

    def scatter(t, b):
        for q in range(2):
            dvec = idx_d[t // 4, pl.ds((t % 4) * 32 + q * 16, 16)]
            pltpu.sync_copy(rows3.at[b].at[pl.ds(q * 16, 16)],
                            accS.at[dvec], add=True)

    for p in range(4):
        gather(p, p)

    def pbody(t, _):
        b = t % 6
        drain(b)

        @pl.when(t + 4 < NT)
        def _g_next():
            gather(t + 4, (t + 4) % 6)

        scatter(t, b)
        return 0

    lax.fori_loop(0, NT, pbody, 0)

    plsc.subcore_barrier()
    pltpu.sync_copy(accS.at[pl.ds(s * SLICE, SLICE)],
                    p_hbm.at[pl.ds(c * NACC + s * SLICE, SLICE)])


@functools.partial(
    pl.kernel,
    out_type=jax.ShapeDtypeStruct((NC * NACC, D), _f32),
    mesh=_mesh(),
    compiler_params=pltpu.CompilerParams(needs_layout_passes=False),
    scratch_types=[
        pltpu.VMEM((CH, 128), _i32),
        pltpu.VMEM((CH, 128), _i32),
        pltpu.VMEM((6, 32, D), _f32),
        pltpu.SemaphoreType.DMA,
        pltpu.VMEM_SHARED((NACC, D), _f32),
    ],
)
def _scatter_kernel(table_hbm, srcp_hbm, dstp_hbm, p_hbm,
                    idx_s, idx_d, rows3, sem, accS):
    _scatter_body(table_hbm, srcp_hbm, dstp_hbm, p_hbm,
                  idx_s, idx_d, rows3, sem, accS)


@functools.partial(
    pl.kernel,
    out_type=jax.ShapeDtypeStruct((NC * NACC, D), _f32),
    mesh=_mesh(),
    compiler_params=pltpu.CompilerParams(needs_layout_passes=False),
    scratch_types=[
        pltpu.VMEM((CH, 128), _i32),
        pltpu.VMEM((CH * 128,), _i32),
        pltpu.VMEM((CH * 128 + 16,), _i32),
        pltpu.VMEM((CH * 128 + 16,), _i32),
        pltpu.VMEM((NACC // 32,), _i32),
        pltpu.VMEM((2, 16, D), _f32),
        pltpu.SemaphoreType.DMA,
        pltpu.VMEM_SHARED((NACC, D), _f32),
    ],
)
def _fscatter_kernel(table_hbm, srcp_hbm, dstf_hbm, bmp_hbm, p_hbm,
                     sraw, draw, sbuf, dbuf, bmp, rows2, sem, accS):
    c = lax.axis_index("c")
    s = lax.axis_index("s")
    w = c * NS + s
    zero = jnp.zeros((16,), _f32)
    lanes = lax.iota(_i32, 16)

    def zrow(i, _):
        rows2[0, i // 8, pl.ds((i % 8) * 16, 16)] = zero
        return 0

    lax.fori_loop(0, 16 * 8, zrow, 0)
    for m in range(SLICE // 16):
        pltpu.sync_copy(rows2.at[0], accS.at[pl.ds(s * SLICE + m * 16, 16)])

    plsc.subcore_barrier()

    pltpu.sync_copy(srcp_hbm.at[pl.ds(w * CH, CH)], sraw)
    pltpu.sync_copy(dstf_hbm.at[pl.ds(w * CH * 128, CH * 128)], draw)
    pltpu.sync_copy(bmp_hbm, bmp)

    dummy_dst = NN + lanes
    zero_i = jnp.zeros((16,), _i32)

    def pfill(i, _):
        sbuf[pl.ds(i * 16, 16)] = zero_i
        dbuf[pl.ds(i * 16, 16)] = dummy_dst
        return 0

    lax.fori_loop(0, (CH * 128 + 16) // 16, pfill, 0)

    def fbody(i, off):
        svec = sraw[i // 8, pl.ds((i % 8) * 16, 16)]
        dvec = draw[pl.ds(i * 16, 16)]
        word = plsc.load_gather(bmp, [lax.shift_right_logical(dvec, 5)])
        bit = lax.shift_right_logical(word, dvec & 31) & 1
        msk = bit != 0
        store_window_s = sbuf.at[pl.ds(off, 16)]
        store_window_d = dbuf.at[pl.ds(off, 16)]
        plsc.store_compressed(store_window_s, svec, mask=msk)
        plsc.store_compressed(store_window_d, dvec, mask=msk)
        cnt = plsc.all_reduce_population_count(msk)
        return off + cnt[0]

    off = lax.fori_loop(0, (CH * 128) // 16, fbody, jnp.int32(0))
    nv = lax.max((off + 15) // 16, 1)

    def gather(t, b):
        svec = sbuf[pl.ds(t * 16, 16)]
        pltpu.async_copy(table_hbm.at[svec], rows2.at[b], sem)

    def drain(b):
        pltpu.make_async_copy(table_hbm.at[pl.ds(0, 16)], rows2.at[b],
                              sem).wait()

    gather(0, 0)

    def pbody(t, _):
        b = t % 2
        drain(b)

        @pl.when(t + 1 < nv)
        def _g_next():
            gather(t + 1, 1 - b)

        dvec = dbuf[pl.ds(t * 16, 16)]
        pltpu.sync_copy(rows2.at[b], accS.at[dvec], add=True)
        return 0

    lax.fori_loop(0, nv, pbody, 0)

    plsc.subcore_barrier()
    pltpu.sync_copy(accS.at[pl.ds(s * SLICE, SLICE)],
                    p_hbm.at[pl.ds(c * NACC + s * SLICE, SLICE)])


@functools.partial(
    pl.kernel,
    out_type=jax.ShapeDtypeStruct((BB, D), _f32),
    mesh=_mesh(),
    compiler_params=pltpu.CompilerParams(needs_layout_passes=False),
    scratch_types=[
        pltpu.VMEM((BB // NW, LL), _i32),
        pltpu.VMEM((BB // NW, D), _f32),
        pltpu.VMEM((2, 64, D), _f32),
        pltpu.SemaphoreType.DMA,
    ],
)
def _text_kernel(eids_hbm, ftab_hbm, tsum_hbm, tidv, tacc, rows2, sem):
    w = lax.axis_index("c") * NS + lax.axis_index("s")
    nseq = BB // NW
    pltpu.sync_copy(eids_hbm.at[pl.ds(w * nseq, nseq)], tidv)

    def tgather(j, h, b):
        pltpu.async_copy(ftab_hbm.at[tidv.at[j, pl.ds(h * 64, 64)]],
                         rows2.at[b], sem)

    def twait(b):
        pltpu.make_async_copy(ftab_hbm.at[pl.ds(0, 64)], rows2.at[b],
                              sem).wait()

    def taccum(b, carry):
        def rbody(i, cin):
            return tuple(cin[k] + rows2[b, i, pl.ds(k * 16, 16)]
                         for k in range(8))

        return lax.fori_loop(0, 64, rbody, carry)

    tgather(0, 0, 0)

    def tbody(j, _):
        twait(0)
        tgather(j, 1, 1)
        accs = taccum(0, tuple(jnp.zeros((16,), _f32) for _ in range(8)))
        twait(1)

        @pl.when(j < nseq - 1)
        def _t_next():
            tgather(j + 1, 0, 0)

        accs = taccum(1, accs)
        for k in range(8):
            tacc[j, pl.ds(k * 16, 16)] = accs[k]
        return 0

    lax.fori_loop(0, nseq, tbody, 0)
    pltpu.sync_copy(tacc, tsum_hbm.at[pl.ds(w * nseq, nseq)])


@functools.partial(
    pl.kernel,
    out_type=jax.ShapeDtypeStruct((BB, D), _f32),
    mesh=_mesh(),
    compiler_params=pltpu.CompilerParams(needs_layout_passes=False),
    scratch_types=[
        pltpu.VMEM((BB // NW,), _i32),
        pltpu.VMEM((BB // NW, D), _f32),
        pltpu.SemaphoreType.DMA,
    ],
)
def _gather_kernel(src_hbm, midx_hbm, out_hbm, idxv, rows, sem):
    w = lax.axis_index("c") * NS + lax.axis_index("s")
    n = BB // NW
    pltpu.sync_copy(midx_hbm.at[pl.ds(w * n, n)], idxv)
    pltpu.async_copy(src_hbm.at[idxv], rows, sem).wait()
    pltpu.sync_copy(rows, out_hbm.at[pl.ds(w * n, n)])


def _prep_body(deg_ref, x_ref, xs0_ref, dinv_ref):
    d = jnp.sum(deg_ref[...], axis=1) + 1.0
    dv = lax.rsqrt(d)
    xs0_ref[...] = x_ref[...] * dv[:, None]
    dinv_ref[...] = jnp.broadcast_to(dv[:, None], dinv_ref.shape)


def _prep_call(degp, x):
    blk = 1000
    return pl.pallas_call(
        _prep_body,
        grid=(NN // blk,),
        in_specs=[pl.BlockSpec((blk, NW), lambda j: (j, 0)),
                  pl.BlockSpec((blk, D), lambda j: (j, 0))],
        out_specs=[pl.BlockSpec((blk, D), lambda j: (j, 0)),
                   pl.BlockSpec((blk, D), lambda j: (j, 0))],
        out_shape=[jax.ShapeDtypeStruct((NN, D), _f32),
                   jax.ShapeDtypeStruct((NN, D), _f32)],
    )(degp, x)


def _mid_body(xs0_ref, p0_ref, p1_ref, dinv_ref, w1_ref, b1_ref, w2_ref,
              out_ref):
    dv = dinv_ref[...]
    agg = (xs0_ref[...] + p0_ref[...] + p1_ref[...]) * dv
    h = jnp.dot(agg, w1_ref[...], preferred_element_type=_f32) + b1_ref[...]
    h = jnp.where(h >= 0, h, 0.01 * h)
    z = jnp.dot(h, w2_ref[...], preferred_element_type=_f32)
    out_ref[...] = z * dv


def _mid_call(xs0, p0, p1, dinvb, W1, b1, W2):
    blk = 1000
    row = pl.BlockSpec((blk, D), lambda j: (j, 0))
    return pl.pallas_call(
        _mid_body,
        grid=(NN // blk,),
        in_specs=[row, row, row, row,
                  pl.BlockSpec((D, 2 * D), lambda j: (0, 0)),
                  pl.BlockSpec((1, 2 * D), lambda j: (0, 0)),
                  pl.BlockSpec((2 * D, D), lambda j: (0, 0))],
        out_specs=row,
        out_shape=jax.ShapeDtypeStruct((NN, D), _f32),
    )(xs0, p0, p1, dinvb, W1, b1, W2)


def _fin_body(xs2_ref, q0_ref, q1_ref, dinv_ref, b2_ref, out_ref):
    blk = out_ref.shape[0]
    j = pl.program_id(0)
    rows = j * blk + lax.broadcasted_iota(_i32, (blk, D), 0)
    v = (xs2_ref[...] + q0_ref[...] + q1_ref[...]) * dinv_ref[...] + b2_ref[...]
    out_ref[...] = jnp.where(rows < NN, v, 0.0)


def _fin_call(xs2, q0, q1, dinvb, b2):
    blk = 1112
    row = pl.BlockSpec((blk, D), lambda j: (j, 0))
    return pl.pallas_call(
        _fin_body,
        grid=(NPAD // blk,),
        in_specs=[row, row, row, row,
                  pl.BlockSpec((1, D), lambda j: (0, 0))],
        out_specs=row,
        out_shape=jax.ShapeDtypeStruct((NPAD, D), _f32),
    )(xs2, q0, q1, dinvb, b2)


def _textmm_body(ts_ref, wt_ref, bt_ref, out_ref):
    t = ts_ref[...] * (1.0 / LL)
    out_ref[...] = (jnp.dot(t, wt_ref[...], preferred_element_type=_f32)
                    + bt_ref[...])


def _textmm_call(tsum, W_text, b_text):
    return pl.pallas_call(
        _textmm_body,
        grid=(1,),
        in_specs=[pl.BlockSpec((BB, D), lambda j: (0, 0)),
                  pl.BlockSpec((D, D), lambda j: (0, 0)),
                  pl.BlockSpec((1, D), lambda j: (0, 0))],
        out_specs=pl.BlockSpec((BB, D), lambda j: (0, 0)),
        out_shape=jax.ShapeDtypeStruct((BB, D), _f32),
    )(tsum, W_text, b_text)


def _bmpack_body(bm_ref, out_ref):
    bits = (bm_ref[...] > 0).astype(_i32)
    shifted = jnp.left_shift(bits, lax.broadcasted_iota(_i32, bits.shape, 1))
    out_ref[...] = jnp.sum(shifted, axis=1, keepdims=True)


def _bmpack_call(bmu):
    n = NACC // 32
    return pl.pallas_call(
        _bmpack_body,
        grid=(1,),
        in_specs=[pl.BlockSpec((n, 32), lambda j: (0, 0))],
        out_specs=pl.BlockSpec((n, 1), lambda j: (0, 0)),
        out_shape=jax.ShapeDtypeStruct((n, 1), _i32),
    )(bmu.reshape(n, 32))


def kernel(e_ids, e_mask, x_graph, edge_index, batch_idx, data_mask,
           ft_table, W_text, b_text, W1, b1, W2, b2):
    x = x_graph.astype(_f32)
    src = edge_index[0].astype(_i32)
    dst = edge_index[1].astype(_i32)
    per_w = NE // NW
    pad_w = EP // NW - per_w
    src_w = jnp.concatenate(
        [src.reshape(NW, per_w), jnp.zeros((NW, pad_w), _i32)], axis=1)
    pad_dst = jnp.broadcast_to(NN + jnp.arange(pad_w, dtype=_i32),
                               (NW, pad_w))
    dst_w = jnp.concatenate([dst.reshape(NW, per_w), pad_dst], axis=1)
    srcp = src_w.reshape(NW * CH, 128)
    dstf = dst_w.reshape(-1)
    dstp = dstf.reshape(NW * CH, 128)

    bidx32 = batch_idx.astype(_i32)
    degp, bmu = _deg_kernel(dstf, bidx32)
    bmp = _bmpack_call(bmu).reshape(-1)
    xs0, dinvb = _prep_call(degp.reshape(NW, DEGW).T, x)
    P = _scatter_kernel(xs0, srcp, dstp)
    tsum = _text_kernel(e_ids.astype(_i32), ft_table.astype(_f32))
    xs2 = _mid_call(xs0, P[:NN], P[NACC:NACC + NN], dinvb,
                    W1.astype(_f32), b1.reshape(1, -1).astype(_f32),
                    W2.astype(_f32))
    Q = _fscatter_kernel(xs2, srcp, dstf, bmp)
    out2p = _fin_call(xs2, Q[:NN], Q[NACC:NACC + NN], dinvb,
                      b2.reshape(1, -1).astype(_f32))
    x_text = _textmm_call(tsum, W_text.astype(_f32),
                          b_text.reshape(1, -1).astype(_f32))
    midx = jnp.where(data_mask, bidx32, NN)
    out_graph = _gather_kernel(out2p, midx)
    return (x_text, out_graph)


# speedup vs baseline: 18.4679x; 1.0566x over previous
import functools

import jax
import jax.numpy as jnp
from jax import lax
from jax.experimental import pallas as pl
from jax.experimental.pallas import tpu as pltpu
from jax.experimental.pallas import tpu_sc as plsc

NN = 10000
NE = 320000
D = 128
BB = 1024
LL = 128
NC, NS = 2, 16
NW = NC * NS
CH = 80
EP = NW * CH * 128
NACC = 10240
SLICE = NACC // NS
NPAD = NN + 8
DEGW = NACC

_mesh = functools.partial(plsc.VectorSubcoreMesh,
                          core_axis_name="c", subcore_axis_name="s")

_f32 = jnp.float32
_i32 = jnp.int32


@functools.partial(
    pl.kernel,
    out_type=(jax.ShapeDtypeStruct((NW * DEGW,), _f32),
              jax.ShapeDtypeStruct((NACC,), _f32)),
    mesh=_mesh(),
    compiler_params=pltpu.CompilerParams(needs_layout_passes=False),
    scratch_types=[
        pltpu.VMEM((CH * 128,), _i32),
        pltpu.VMEM((DEGW,), _f32),
        pltpu.VMEM((BB,), _i32),
    ],
)
def _deg_kernel(dstf_hbm, bidx_hbm, out_hbm, bmu_hbm, dstbuf, acc, bidx):
    w = lax.axis_index("c") * NS + lax.axis_index("s")
    pltpu.sync_copy(dstf_hbm.at[pl.ds(w * CH * 128, CH * 128)], dstbuf)
    zero = jnp.zeros((16,), _f32)

    def zbody(i, _):
        acc[pl.ds(i * 16, 16)] = zero
        return 0

    lax.fori_loop(0, DEGW // 16, zbody, 0)
    ones = jnp.ones((16,), _f32)

    def body(i, _):
        idx = dstbuf[pl.ds(i * 16, 16)]
        plsc.addupdate_scatter(acc, [idx], ones)
        return 0

    lax.fori_loop(0, (CH * 128) // 16, body, 0)
    pltpu.sync_copy(acc, out_hbm.at[pl.ds(w * DEGW, DEGW)])

    @pl.when(w == 0)
    def _build_bitmap():
        def z2(i, _):
            acc[pl.ds(i * 16, 16)] = zero
            return 0

        lax.fori_loop(0, NACC // 16, z2, 0)
        pltpu.sync_copy(bidx_hbm, bidx)

        def sbody(i, _):
            idx = bidx[pl.ds(i * 16, 16)]
            plsc.store_scatter(acc, [idx], ones)
            return 0

        lax.fori_loop(0, BB // 16, sbody, 0)
        pltpu.sync_copy(acc.at[pl.ds(0, NACC)], bmu_hbm)


def _scatter_body(table_hbm, srcp_hbm, dstp_hbm, p_hbm,
                  idx_s, idx_d, rows3, sem, accS):
    c = lax.axis_index("c")
    s = lax.axis_index("s")
    w = c * NS + s
    zero = jnp.zeros((16,), _f32)

    def zbody(i, _):
        rows3[0, i // 8, pl.ds((i % 8) * 16, 16)] = zero
        return 0

    lax.fori_loop(0, 32 * 8, zbody, 0)
    for m in range(SLICE // 32):
        pltpu.sync_copy(rows3.at[0], accS.at[pl.ds(s * SLICE + m * 32, 32)])

    plsc.subcore_barrier()

    pltpu.sync_copy(srcp_hbm.at[pl.ds(w * CH, CH)], idx_s)
    pltpu.sync_copy(dstp_hbm.at[pl.ds(w * CH, CH)], idx_d)

    NT = CH * 4

    def gather(t, b):
        pltpu.async_copy(
            table_hbm.at[idx_s.at[t // 4, pl.ds((t % 4) * 32, 32)]],
            rows3.at[b], sem)

    def drain(b):
        pltpu.make_async_copy(table_hbm.at[pl.ds(0, 32)], rows3.at[b],
                              sem).wait()

    def scatter(t, b):
        for q in range(2):
            dvec = idx_d[t // 4, pl.ds((t % 4) * 32 + q * 16, 16)]
            pltpu.sync_copy(rows3.at[b].at[pl.ds(q * 16, 16)],
                            accS.at[dvec], add=True)

    for p in range(4):
        gather(p, p)

    def pbody(t, _):
        b = t % 6
        drain(b)

        @pl.when(t + 4 < NT)
        def _g_next():
            gather(t + 4, (t + 4) % 6)

        scatter(t, b)
        return 0

    lax.fori_loop(0, NT, pbody, 0)

    plsc.subcore_barrier()
    pltpu.sync_copy(accS.at[pl.ds(s * SLICE, SLICE)],
                    p_hbm.at[pl.ds(c * NACC + s * SLICE, SLICE)])


@functools.partial(
    pl.kernel,
    out_type=jax.ShapeDtypeStruct((NC * NACC, D), _f32),
    mesh=_mesh(),
    compiler_params=pltpu.CompilerParams(needs_layout_passes=False),
    scratch_types=[
        pltpu.VMEM((CH, 128), _i32),
        pltpu.VMEM((CH, 128), _i32),
        pltpu.VMEM((6, 32, D), _f32),
        pltpu.SemaphoreType.DMA,
        pltpu.VMEM_SHARED((NACC, D), _f32),
    ],
)
def _scatter_kernel(table_hbm, srcp_hbm, dstp_hbm, p_hbm,
                    idx_s, idx_d, rows3, sem, accS):
    _scatter_body(table_hbm, srcp_hbm, dstp_hbm, p_hbm,
                  idx_s, idx_d, rows3, sem, accS)


@functools.partial(
    pl.kernel,
    out_type=jax.ShapeDtypeStruct((NC * NACC, D), _f32),
    mesh=_mesh(),
    compiler_params=pltpu.CompilerParams(needs_layout_passes=False),
    scratch_types=[
        pltpu.VMEM((CH, 128), _i32),
        pltpu.VMEM((CH * 128,), _i32),
        pltpu.VMEM((CH * 128 + 16,), _i32),
        pltpu.VMEM((CH * 128 + 16,), _i32),
        pltpu.VMEM((NACC // 32,), _i32),
        pltpu.VMEM((2, 16, D), _f32),
        pltpu.SemaphoreType.DMA,
        pltpu.VMEM_SHARED((NACC, D), _f32),
    ],
)
def _fscatter_kernel(table_hbm, srcp_hbm, dstf_hbm, bmp_hbm, p_hbm,
                     sraw, draw, sbuf, dbuf, bmp, rows2, sem, accS):
    c = lax.axis_index("c")
    s = lax.axis_index("s")
    w = c * NS + s
    zero = jnp.zeros((16,), _f32)
    lanes = lax.iota(_i32, 16)

    def zrow(i, _):
        rows2[0, i // 8, pl.ds((i % 8) * 16, 16)] = zero
        return 0

    lax.fori_loop(0, 16 * 8, zrow, 0)
    for m in range(SLICE // 16):
        pltpu.sync_copy(rows2.at[0], accS.at[pl.ds(s * SLICE + m * 16, 16)])

    plsc.subcore_barrier()

    pltpu.sync_copy(srcp_hbm.at[pl.ds(w * CH, CH)], sraw)
    pltpu.sync_copy(dstf_hbm.at[pl.ds(w * CH * 128, CH * 128)], draw)
    pltpu.sync_copy(bmp_hbm, bmp)

    dummy_dst = NN + lanes
    zero_i = jnp.zeros((16,), _i32)

    def pfill(i, _):
        sbuf[pl.ds(i * 16, 16)] = zero_i
        dbuf[pl.ds(i * 16, 16)] = dummy_dst
        return 0

    lax.fori_loop(0, (CH * 128 + 16) // 16, pfill, 0)

    def fbody(i, off):
        svec = sraw[i // 8, pl.ds((i % 8) * 16, 16)]
        dvec = draw[pl.ds(i * 16, 16)]
        word = plsc.load_gather(bmp, [lax.shift_right_logical(dvec, 5)])
        bit = lax.shift_right_logical(word, dvec & 31) & 1
        msk = bit != 0
        store_window_s = sbuf.at[pl.ds(off, 16)]
        store_window_d = dbuf.at[pl.ds(off, 16)]
        plsc.store_compressed(store_window_s, svec, mask=msk)
        plsc.store_compressed(store_window_d, dvec, mask=msk)
        cnt = plsc.all_reduce_population_count(msk)
        return off + cnt[0]

    off = lax.fori_loop(0, (CH * 128) // 16, fbody, jnp.int32(0))
    nv = lax.max((off + 15) // 16, 1)

    def gather(t, b):
        svec = sbuf[pl.ds(t * 16, 16)]
        pltpu.async_copy(table_hbm.at[svec], rows2.at[b], sem)

    def drain(b):
        pltpu.make_async_copy(table_hbm.at[pl.ds(0, 16)], rows2.at[b],
                              sem).wait()

    gather(0, 0)

    def pbody(t, _):
        b = t % 2
        drain(b)

        @pl.when(t + 1 < nv)
        def _g_next():
            gather(t + 1, 1 - b)

        dvec = dbuf[pl.ds(t * 16, 16)]
        pltpu.sync_copy(rows2.at[b], accS.at[dvec], add=True)
        return 0

    lax.fori_loop(0, nv, pbody, 0)

    plsc.subcore_barrier()
    pltpu.sync_copy(accS.at[pl.ds(s * SLICE, SLICE)],
                    p_hbm.at[pl.ds(c * NACC + s * SLICE, SLICE)])


@functools.partial(
    pl.kernel,
    out_type=jax.ShapeDtypeStruct((BB, D), _f32),
    mesh=_mesh(),
    compiler_params=pltpu.CompilerParams(needs_layout_passes=False),
    scratch_types=[
        pltpu.VMEM((BB // NW, LL), _i32),
        pltpu.VMEM((BB // NW, D), _f32),
        pltpu.VMEM((4, 64, D), _f32),
        pltpu.SemaphoreType.DMA,
    ],
)
def _text_kernel(eids_hbm, ftab_hbm, tsum_hbm, tidv, tacc, rows4, sem):
    w = lax.axis_index("c") * NS + lax.axis_index("s")
    nseq = BB // NW
    pltpu.sync_copy(eids_hbm.at[pl.ds(w * nseq, nseq)], tidv)

    NT = nseq * 2

    def tgather(t, b):
        pltpu.async_copy(
            ftab_hbm.at[tidv.at[t // 2, pl.ds((t % 2) * 64, 64)]],
            rows4.at[b], sem)

    def twait(b):
        pltpu.make_async_copy(ftab_hbm.at[pl.ds(0, 64)], rows4.at[b],
                              sem).wait()

    zeros8 = tuple(jnp.zeros((16,), _f32) for _ in range(8))

    for p in range(3):
        tgather(p, p)

    def tbody(t, carry):
        b = t % 4
        twait(b)

        @pl.when(t + 3 < NT)
        def _t_next():
            tgather(t + 3, (t + 3) % 4)

        fresh = t % 2 == 0
        base = tuple(jnp.where(fresh, z, c) for z, c in zip(zeros8, carry))

        def rbody(i, cin):
            return tuple(cin[k] + rows4[b, i, pl.ds(k * 16, 16)]
                         for k in range(8))

        accs = lax.fori_loop(0, 64, rbody, base)

        @pl.when(t % 2 == 1)
        def _store():
            for k in range(8):
                tacc[t // 2, pl.ds(k * 16, 16)] = accs[k]

        return accs

    lax.fori_loop(0, NT, tbody, zeros8)
    pltpu.sync_copy(tacc, tsum_hbm.at[pl.ds(w * nseq, nseq)])


@functools.partial(
    pl.kernel,
    out_type=jax.ShapeDtypeStruct((BB, D), _f32),
    mesh=_mesh(),
    compiler_params=pltpu.CompilerParams(needs_layout_passes=False),
    scratch_types=[
        pltpu.VMEM((BB // NW,), _i32),
        pltpu.VMEM((BB // NW, D), _f32),
        pltpu.SemaphoreType.DMA,
    ],
)
def _gather_kernel(src_hbm, midx_hbm, out_hbm, idxv, rows, sem):
    w = lax.axis_index("c") * NS + lax.axis_index("s")
    n = BB // NW
    pltpu.sync_copy(midx_hbm.at[pl.ds(w * n, n)], idxv)
    pltpu.async_copy(src_hbm.at[idxv], rows, sem).wait()
    pltpu.sync_copy(rows, out_hbm.at[pl.ds(w * n, n)])


def _prep_body(deg_ref, x_ref, xs0_ref, dinv_ref):
    d = jnp.sum(deg_ref[...], axis=1) + 1.0
    dv = lax.rsqrt(d)
    xs0_ref[...] = x_ref[...] * dv[:, None]
    dinv_ref[...] = jnp.broadcast_to(dv[:, None], dinv_ref.shape)


def _prep_call(degp, x):
    blk = 1000
    return pl.pallas_call(
        _prep_body,
        grid=(NN // blk,),
        in_specs=[pl.BlockSpec((blk, NW), lambda j: (j, 0)),
                  pl.BlockSpec((blk, D), lambda j: (j, 0))],
        out_specs=[pl.BlockSpec((blk, D), lambda j: (j, 0)),
                   pl.BlockSpec((blk, D), lambda j: (j, 0))],
        out_shape=[jax.ShapeDtypeStruct((NN, D), _f32),
                   jax.ShapeDtypeStruct((NN, D), _f32)],
    )(degp, x)


def _mid_body(xs0_ref, p0_ref, p1_ref, dinv_ref, w1_ref, b1_ref, w2_ref,
              out_ref):
    dv = dinv_ref[...]
    agg = (xs0_ref[...] + p0_ref[...] + p1_ref[...]) * dv
    h = jnp.dot(agg, w1_ref[...], preferred_element_type=_f32) + b1_ref[...]
    h = jnp.where(h >= 0, h, 0.01 * h)
    z = jnp.dot(h, w2_ref[...], preferred_element_type=_f32)
    out_ref[...] = z * dv


def _mid_call(xs0, p0, p1, dinvb, W1, b1, W2):
    blk = 1000
    row = pl.BlockSpec((blk, D), lambda j: (j, 0))
    return pl.pallas_call(
        _mid_body,
        grid=(NN // blk,),
        in_specs=[row, row, row, row,
                  pl.BlockSpec((D, 2 * D), lambda j: (0, 0)),
                  pl.BlockSpec((1, 2 * D), lambda j: (0, 0)),
                  pl.BlockSpec((2 * D, D), lambda j: (0, 0))],
        out_specs=row,
        out_shape=jax.ShapeDtypeStruct((NN, D), _f32),
    )(xs0, p0, p1, dinvb, W1, b1, W2)


def _fin_body(xs2_ref, q0_ref, q1_ref, dinv_ref, b2_ref, out_ref):
    blk = out_ref.shape[0]
    j = pl.program_id(0)
    rows = j * blk + lax.broadcasted_iota(_i32, (blk, D), 0)
    v = (xs2_ref[...] + q0_ref[...] + q1_ref[...]) * dinv_ref[...] + b2_ref[...]
    out_ref[...] = jnp.where(rows < NN, v, 0.0)


def _fin_call(xs2, q0, q1, dinvb, b2):
    blk = 1112
    row = pl.BlockSpec((blk, D), lambda j: (j, 0))
    return pl.pallas_call(
        _fin_body,
        grid=(NPAD // blk,),
        in_specs=[row, row, row, row,
                  pl.BlockSpec((1, D), lambda j: (0, 0))],
        out_specs=row,
        out_shape=jax.ShapeDtypeStruct((NPAD, D), _f32),
    )(xs2, q0, q1, dinvb, b2)


def _textmm_body(ts_ref, wt_ref, bt_ref, out_ref):
    t = ts_ref[...] * (1.0 / LL)
    out_ref[...] = (jnp.dot(t, wt_ref[...], preferred_element_type=_f32)
                    + bt_ref[...])


def _textmm_call(tsum, W_text, b_text):
    return pl.pallas_call(
        _textmm_body,
        grid=(1,),
        in_specs=[pl.BlockSpec((BB, D), lambda j: (0, 0)),
                  pl.BlockSpec((D, D), lambda j: (0, 0)),
                  pl.BlockSpec((1, D), lambda j: (0, 0))],
        out_specs=pl.BlockSpec((BB, D), lambda j: (0, 0)),
        out_shape=jax.ShapeDtypeStruct((BB, D), _f32),
    )(tsum, W_text, b_text)


def _bmpack_body(bm_ref, out_ref):
    bits = (bm_ref[...] > 0).astype(_i32)
    shifted = jnp.left_shift(bits, lax.broadcasted_iota(_i32, bits.shape, 1))
    out_ref[...] = jnp.sum(shifted, axis=1, keepdims=True)


def _bmpack_call(bmu):
    n = NACC // 32
    return pl.pallas_call(
        _bmpack_body,
        grid=(1,),
        in_specs=[pl.BlockSpec((n, 32), lambda j: (0, 0))],
        out_specs=pl.BlockSpec((n, 1), lambda j: (0, 0)),
        out_shape=jax.ShapeDtypeStruct((n, 1), _i32),
    )(bmu.reshape(n, 32))


def kernel(e_ids, e_mask, x_graph, edge_index, batch_idx, data_mask,
           ft_table, W_text, b_text, W1, b1, W2, b2):
    x = x_graph.astype(_f32)
    src = edge_index[0].astype(_i32)
    dst = edge_index[1].astype(_i32)
    per_w = NE // NW
    pad_w = EP // NW - per_w
    src_w = jnp.concatenate(
        [src.reshape(NW, per_w), jnp.zeros((NW, pad_w), _i32)], axis=1)
    pad_dst = jnp.broadcast_to(NN + jnp.arange(pad_w, dtype=_i32),
                               (NW, pad_w))
    dst_w = jnp.concatenate([dst.reshape(NW, per_w), pad_dst], axis=1)
    srcp = src_w.reshape(NW * CH, 128)
    dstf = dst_w.reshape(-1)
    dstp = dstf.reshape(NW * CH, 128)

    bidx32 = batch_idx.astype(_i32)
    degp, bmu = _deg_kernel(dstf, bidx32)
    bmp = _bmpack_call(bmu).reshape(-1)
    xs0, dinvb = _prep_call(degp.reshape(NW, DEGW).T, x)
    P = _scatter_kernel(xs0, srcp, dstp)
    tsum = _text_kernel(e_ids.astype(_i32), ft_table.astype(_f32))
    xs2 = _mid_call(xs0, P[:NN], P[NACC:NACC + NN], dinvb,
                    W1.astype(_f32), b1.reshape(1, -1).astype(_f32),
                    W2.astype(_f32))
    Q = _fscatter_kernel(xs2, srcp, dstf, bmp)
    out2p = _fin_call(xs2, Q[:NN], Q[NACC:NACC + NN], dinvb,
                      b2.reshape(1, -1).astype(_f32))
    x_text = _textmm_call(tsum, W_text.astype(_f32),
                          b_text.reshape(1, -1).astype(_f32))
    midx = jnp.where(data_mask, bidx32, NN)
    out_graph = _gather_kernel(out2p, midx)
    return (x_text, out_graph)
